# Initial kernel scaffold; baseline (speedup 1.0000x reference)
import jax
import jax.numpy as jnp
from jax.experimental import pallas as pl


def kernel(node_features, params, edges, input_node_indices):
    raise NotImplementedError("write your pallas kernel here")


# speedup vs baseline: 3.3188x; 3.3188x over previous
import functools

import jax
import jax.numpy as jnp
from jax import lax
from jax.experimental import pallas as pl
from jax.experimental.pallas import tpu as pltpu
from jax.experimental.pallas import tpu_sc as plsc

_NC = 2
_NS = 16
_NW = _NC * _NS

_SQRT2 = 1.4142135623730951


def _gelu(x):
    return x * 0.5 * (1.0 + lax.erf(x / _SQRT2))


def _bn_params(layer):
    s = layer["gamma"] / jnp.sqrt(layer["mvar"] + 1e-3)
    t = layer["beta"] - layer["mmean"] * s
    return [s.reshape(1, -1), t.reshape(1, -1), layer["W"],
            layer["b"].reshape(1, -1)]


def _layer(x, s, t, w, b):
    return _gelu(jnp.dot(x * s[...] + t[...], w[...],
                         preferred_element_type=jnp.float32) + b[...])


def _wspecs(arrs):
    return [pl.BlockSpec(a.shape, lambda i, r=len(a.shape): (0,) * r)
            for a in arrs]


def _tc1(nf, pre, prep, inv_e, n, br):
    def body(nf_r, s1, t1, w1, b1, s2, t2, w2, b2,
             p1s, p1t, p1w, p1b, p2s, p2t, p2w, p2b, x_r, m_r):
        x = _layer(_layer(nf_r[...], s1, t1, w1, b1), s2, t2, w2, b2)
        x_r[...] = x
        p = _layer(_layer(x, p1s, p1t, p1w, p1b), p2s, p2t, p2w, p2b)
        m_r[...] = p * inv_e

    d = nf.shape[1]
    ws = pre[0] + pre[1] + prep[0] + prep[1]
    return pl.pallas_call(
        body,
        grid=(n // br,),
        in_specs=[pl.BlockSpec((br, d), lambda i: (i, 0))] + _wspecs(ws),
        out_specs=[pl.BlockSpec((br, 32), lambda i: (i, 0)),
                   pl.BlockSpec((br, 32), lambda i: (i, 0))],
        out_shape=[jax.ShapeDtypeStruct((n, 32), jnp.float32),
                   jax.ShapeDtypeStruct((n, 32), jnp.float32)],
        compiler_params=pltpu.CompilerParams(
            dimension_semantics=("parallel",)),
    )(nf, *ws)


def _tc_combine(x, sums, cnts, upd, prep, inv_e, n, br):
    with_prep = prep is not None

    def body(*refs):
        if with_prep:
            (x_r, s_r, c_r, u1s, u1t, u1w, u1b, u2s, u2t, u2w, u2b,
             p1s, p1t, p1w, p1b, p2s, p2t, p2w, p2b, xa_r, m_r) = refs
        else:
            (x_r, s_r, c_r, u1s, u1t, u1w, u1b, u2s, u2t, u2w, u2b,
             xa_r) = refs
        counts = jnp.maximum(c_r[0, :, 0:1] + c_r[1, :, 0:1], 1.0)
        agg = (s_r[0] + s_r[1]) / counts
        x = x_r[...]
        h = jnp.concatenate([x, agg], axis=1)
        u = _layer(_layer(h, u1s, u1t, u1w, u1b), u2s, u2t, u2w, u2b)
        nrm = jnp.sqrt(jnp.maximum(jnp.sum(u * u, axis=1, keepdims=True),
                                   1e-12))
        xa = u / nrm + x
        xa_r[...] = xa
        if with_prep:
            p = _layer(_layer(xa, p1s, p1t, p1w, p1b), p2s, p2t, p2w, p2b)
            m_r[...] = p * inv_e

    ws = upd[0] + upd[1] + (prep[0] + prep[1] if with_prep else [])
    in_specs = [
        pl.BlockSpec((br, 32), lambda i: (i, 0)),
        pl.BlockSpec((2, br, 32), lambda i: (0, i, 0)),
        pl.BlockSpec((2, br, 16), lambda i: (0, i, 0)),
    ] + _wspecs(ws)
    nout = 2 if with_prep else 1
    out_specs = [pl.BlockSpec((br, 32), lambda i: (i, 0))] * nout
    out_shape = [jax.ShapeDtypeStruct((n, 32), jnp.float32)] * nout

    res = pl.pallas_call(
        body,
        grid=(n // br,),
        in_specs=in_specs,
        out_specs=out_specs,
        out_shape=out_shape,
        compiler_params=pltpu.CompilerParams(
            dimension_semantics=("parallel",)),
    )(x, sums, cnts, *ws)
    return res


def _tc4(emb, post, wl, bl):
    def body(e_r, q1s, q1t, q1w, q1b, q2s, q2t, q2w, q2b, wlr, blr, o_r):
        p = _layer(_layer(e_r[...], q1s, q1t, q1w, q1b), q2s, q2t, q2w, q2b)
        o_r[...] = jnp.dot(p, wlr[...],
                           preferred_element_type=jnp.float32) + blr[...]

    b = emb.shape[0]
    return pl.pallas_call(
        body,
        out_shape=jax.ShapeDtypeStruct((b, wl.shape[1]), jnp.float32),
    )(emb, *post[0], *post[1], wl, bl)


def _row_partition(n):
    rpt = ((n // _NS + 7) // 8) * 8
    last = n - (_NS - 1) * rpt
    assert last > 0 and last % 8 == 0 and rpt % 8 == 0
    return rpt, last


def _sc_counts(dst, zeros16, ones16, n, e, k):
    epw = e // _NW
    rpt, last = _row_partition(n)
    mesh = plsc.VectorSubcoreMesh(core_axis_name="c", subcore_axis_name="s")

    @functools.partial(
        pl.kernel,
        out_type=jax.ShapeDtypeStruct((_NC, n, 16), jnp.float32),
        mesh=mesh,
        compiler_params=pltpu.CompilerParams(use_tc_tiling_on_sc=False),
        scratch_types=[
            pltpu.VMEM_SHARED((n, 16), jnp.float32),
            pltpu.VMEM((k,), jnp.int32),
            pltpu.VMEM((k, 16), jnp.float32),
        ],
    )
    def kern(dst_hbm, z_hbm, o_hbm, out_hbm, acc, didx, ones_v):
        c = lax.axis_index("c")
        s = lax.axis_index("s")
        pltpu.sync_copy(o_hbm, ones_v)

        @pl.when(s < _NS - 1)
        def _():
            pltpu.sync_copy(z_hbm, acc.at[pl.ds(s * rpt, rpt)])

        @pl.when(s == _NS - 1)
        def _():
            pltpu.sync_copy(z_hbm.at[pl.ds(0, last)],
                            acc.at[pl.ds(s * rpt, last)])

        plsc.subcore_barrier()
        base = c * (e // _NC) + s * epw

        def body(i, carry):
            pltpu.sync_copy(dst_hbm.at[pl.ds(base + i * k, k)], didx)
            pltpu.sync_copy(ones_v, acc.at[didx], add=True)
            return carry

        lax.fori_loop(0, epw // k, body, 0)
        plsc.subcore_barrier()

        @pl.when(s < _NS - 1)
        def _():
            pltpu.sync_copy(acc.at[pl.ds(s * rpt, rpt)],
                            out_hbm.at[c, pl.ds(s * rpt, rpt)])

        @pl.when(s == _NS - 1)
        def _():
            pltpu.sync_copy(acc.at[pl.ds(s * rpt, last)],
                            out_hbm.at[c, pl.ds(s * rpt, last)])

    return kern(dst, zeros16, ones16)


def _sc_scatter(m, src, dst, zeros32, n, e, k):
    epw = e // _NW
    rpt, last = _row_partition(n)
    mesh = plsc.VectorSubcoreMesh(core_axis_name="c", subcore_axis_name="s")

    @functools.partial(
        pl.kernel,
        out_type=jax.ShapeDtypeStruct((_NC, n, 32), jnp.float32),
        mesh=mesh,
        compiler_params=pltpu.CompilerParams(use_tc_tiling_on_sc=False),
        scratch_types=[
            pltpu.VMEM_SHARED((n, 32), jnp.float32),
            pltpu.VMEM((k,), jnp.int32),
            pltpu.VMEM((k,), jnp.int32),
            pltpu.VMEM((k, 32), jnp.float32),
            pltpu.SemaphoreType.DMA,
        ],
    )
    def kern(m_hbm, src_hbm, dst_hbm, z_hbm, out_hbm,
             acc, sidx, didx, rows, sem):
        c = lax.axis_index("c")
        s = lax.axis_index("s")

        @pl.when(s < _NS - 1)
        def _():
            pltpu.sync_copy(z_hbm, acc.at[pl.ds(s * rpt, rpt)])

        @pl.when(s == _NS - 1)
        def _():
            pltpu.sync_copy(z_hbm.at[pl.ds(0, last)],
                            acc.at[pl.ds(s * rpt, last)])

        plsc.subcore_barrier()
        base = c * (e // _NC) + s * epw

        def body(i, carry):
            off = base + i * k
            pltpu.sync_copy(src_hbm.at[pl.ds(off, k)], sidx)
            pltpu.sync_copy(dst_hbm.at[pl.ds(off, k)], didx)
            pltpu.async_copy(m_hbm.at[sidx], rows, sem).wait()
            pltpu.sync_copy(rows, acc.at[didx], add=True)
            return carry

        lax.fori_loop(0, epw // k, body, 0)
        plsc.subcore_barrier()

        @pl.when(s < _NS - 1)
        def _():
            pltpu.sync_copy(acc.at[pl.ds(s * rpt, rpt)],
                            out_hbm.at[c, pl.ds(s * rpt, rpt)])

        @pl.when(s == _NS - 1)
        def _():
            pltpu.sync_copy(acc.at[pl.ds(s * rpt, last)],
                            out_hbm.at[c, pl.ds(s * rpt, last)])

    return kern(m, src, dst, zeros32)


def _sc_gather(xb, idx, n, b):
    bpw = b // _NW
    mesh = plsc.VectorSubcoreMesh(core_axis_name="c", subcore_axis_name="s")

    @functools.partial(
        pl.kernel,
        out_type=jax.ShapeDtypeStruct((b, 32), jnp.float32),
        mesh=mesh,
        compiler_params=pltpu.CompilerParams(use_tc_tiling_on_sc=False),
        scratch_types=[
            pltpu.VMEM((bpw,), jnp.int32),
            pltpu.VMEM((bpw, 32), jnp.float32),
            pltpu.SemaphoreType.DMA,
        ],
    )
    def kern(x_hbm, idx_hbm, out_hbm, idxv, rows, sem):
        c = lax.axis_index("c")
        s = lax.axis_index("s")
        base = (s * _NC + c) * bpw
        pltpu.sync_copy(idx_hbm.at[pl.ds(base, bpw)], idxv)
        pltpu.async_copy(x_hbm.at[idxv], rows, sem).wait()
        pltpu.sync_copy(rows, out_hbm.at[pl.ds(base, bpw)])

    return kern(xb, idx)


def kernel(node_features, params, edges, input_node_indices):
    n, d = node_features.shape
    e = edges.shape[1]
    inv_e = 1.0 / float(e)
    br = 2000
    k = 40

    ffns = {name: [_bn_params(l) for l in params[name]]
            for name in ("pre", "conv1_prep", "conv1_upd",
                         "conv2_prep", "conv2_upd", "post")}

    dst = edges[0]
    src = edges[1]
    rpt, _ = _row_partition(n)
    zeros32 = jnp.zeros((rpt, 32), jnp.float32)
    zeros16 = jnp.zeros((rpt, 16), jnp.float32)
    ones16 = jnp.ones((k, 16), jnp.float32)

    cnts = _sc_counts(dst, zeros16, ones16, n, e, k)

    x, m1 = _tc1(node_features, ffns["pre"], ffns["conv1_prep"],
                 inv_e, n, br)
    sums1 = _sc_scatter(m1, src, dst, zeros32, n, e, k)
    xa, m2 = _tc_combine(x, sums1, cnts, ffns["conv1_upd"],
                         ffns["conv2_prep"], inv_e, n, br)
    sums2 = _sc_scatter(m2, src, dst, zeros32, n, e, k)
    xb = _tc_combine(xa, sums2, cnts, ffns["conv2_upd"],
                     None, inv_e, n, br)[0]
    emb = _sc_gather(xb, input_node_indices, n, input_node_indices.shape[0])
    return _tc4(emb, ffns["post"], params["Wl"], params["bl"].reshape(1, -1))


# speedup vs baseline: 12.3534x; 3.7222x over previous
import functools

import jax
import jax.numpy as jnp
from jax import lax
from jax.experimental import pallas as pl
from jax.experimental.pallas import tpu as pltpu
from jax.experimental.pallas import tpu_sc as plsc

_NC = 2
_NS = 16
_NW = _NC * _NS

_SQRT2 = 1.4142135623730951


def _gelu(x):
    return x * 0.5 * (1.0 + lax.erf(x / _SQRT2))


def _bn_params(layer):
    s = layer["gamma"] / jnp.sqrt(layer["mvar"] + 1e-3)
    t = layer["beta"] - layer["mmean"] * s
    return [s.reshape(1, -1), t.reshape(1, -1), layer["W"],
            layer["b"].reshape(1, -1)]


def _layer(x, s, t, w, b):
    return _gelu(jnp.dot(x * s[...] + t[...], w[...],
                         preferred_element_type=jnp.float32) + b[...])


def _wspecs(arrs):
    return [pl.BlockSpec(a.shape, lambda i, r=len(a.shape): (0,) * r)
            for a in arrs]


def _tc1(nf, pre, prep, inv_e, n, br):
    def body(nf_r, s1, t1, w1, b1, s2, t2, w2, b2,
             p1s, p1t, p1w, p1b, p2s, p2t, p2w, p2b, x_r, m_r):
        x = _layer(_layer(nf_r[...], s1, t1, w1, b1), s2, t2, w2, b2)
        x_r[...] = x
        p = _layer(_layer(x, p1s, p1t, p1w, p1b), p2s, p2t, p2w, p2b)
        m_r[...] = p * inv_e

    d = nf.shape[1]
    ws = pre[0] + pre[1] + prep[0] + prep[1]
    return pl.pallas_call(
        body,
        grid=(n // br,),
        in_specs=[pl.BlockSpec((br, d), lambda i: (i, 0))] + _wspecs(ws),
        out_specs=[pl.BlockSpec((br, 32), lambda i: (i, 0)),
                   pl.BlockSpec((br, 32), lambda i: (i, 0))],
        out_shape=[jax.ShapeDtypeStruct((n, 32), jnp.float32),
                   jax.ShapeDtypeStruct((n, 32), jnp.float32)],
        compiler_params=pltpu.CompilerParams(
            dimension_semantics=("parallel",)),
    )(nf, *ws)


def _tc_combine(x, sums, cnts, upd, prep, inv_e, n, br):
    with_prep = prep is not None

    def body(*refs):
        if with_prep:
            (x_r, s_r, c_r, u1s, u1t, u1w, u1b, u2s, u2t, u2w, u2b,
             p1s, p1t, p1w, p1b, p2s, p2t, p2w, p2b, xa_r, m_r) = refs
        else:
            (x_r, s_r, c_r, u1s, u1t, u1w, u1b, u2s, u2t, u2w, u2b,
             xa_r) = refs
        counts = jnp.maximum(c_r[0, :, 0:1] + c_r[1, :, 0:1], 1.0)
        agg = (s_r[0] + s_r[1]) / counts
        x = x_r[...]
        h = jnp.concatenate([x, agg], axis=1)
        u = _layer(_layer(h, u1s, u1t, u1w, u1b), u2s, u2t, u2w, u2b)
        nrm = jnp.sqrt(jnp.maximum(jnp.sum(u * u, axis=1, keepdims=True),
                                   1e-12))
        xa = u / nrm + x
        xa_r[...] = xa
        if with_prep:
            p = _layer(_layer(xa, p1s, p1t, p1w, p1b), p2s, p2t, p2w, p2b)
            m_r[...] = p * inv_e

    ws = upd[0] + upd[1] + (prep[0] + prep[1] if with_prep else [])
    in_specs = [
        pl.BlockSpec((br, 32), lambda i: (i, 0)),
        pl.BlockSpec((2, br, 32), lambda i: (0, i, 0)),
        pl.BlockSpec((2, br, 16), lambda i: (0, i, 0)),
    ] + _wspecs(ws)
    nout = 2 if with_prep else 1
    out_specs = [pl.BlockSpec((br, 32), lambda i: (i, 0))] * nout
    out_shape = [jax.ShapeDtypeStruct((n, 32), jnp.float32)] * nout

    res = pl.pallas_call(
        body,
        grid=(n // br,),
        in_specs=in_specs,
        out_specs=out_specs,
        out_shape=out_shape,
        compiler_params=pltpu.CompilerParams(
            dimension_semantics=("parallel",)),
    )(x, sums, cnts, *ws)
    return res


def _tc4(emb, post, wl, bl):
    def body(e_r, q1s, q1t, q1w, q1b, q2s, q2t, q2w, q2b, wlr, blr, o_r):
        p = _layer(_layer(e_r[...], q1s, q1t, q1w, q1b), q2s, q2t, q2w, q2b)
        o_r[...] = jnp.dot(p, wlr[...],
                           preferred_element_type=jnp.float32) + blr[...]

    b = emb.shape[0]
    return pl.pallas_call(
        body,
        out_shape=jax.ShapeDtypeStruct((b, wl.shape[1]), jnp.float32),
    )(emb, *post[0], *post[1], wl, bl)


def _row_partition(n):
    rpt = ((n // _NS + 7) // 8) * 8
    last = n - (_NS - 1) * rpt
    assert last > 0 and last % 8 == 0 and rpt % 8 == 0
    return rpt, last


def _sc_counts(dst2, zeros16, ones_hbm, n_acc, jpt):
    rpt, last = _row_partition(n_acc)
    mesh = plsc.VectorSubcoreMesh(core_axis_name="c", subcore_axis_name="s")

    @functools.partial(
        pl.kernel,
        out_type=jax.ShapeDtypeStruct((_NC, n_acc, 16), jnp.float32),
        mesh=mesh,
        compiler_params=pltpu.CompilerParams(use_tc_tiling_on_sc=False),
        scratch_types=[
            pltpu.VMEM_SHARED((n_acc, 16), jnp.float32),
            pltpu.VMEM((jpt, 128), jnp.int32),
            pltpu.VMEM((128, 16), jnp.float32),
        ],
    )
    def kern(dst_hbm, z_hbm, o_hbm, out_hbm, acc, didx, ones_v):
        c = lax.axis_index("c")
        s = lax.axis_index("s")
        w = c * _NS + s
        pltpu.sync_copy(o_hbm, ones_v)
        pltpu.sync_copy(dst_hbm.at[pl.ds(w * jpt, jpt)], didx)

        @pl.when(s < _NS - 1)
        def _():
            pltpu.sync_copy(z_hbm, acc.at[pl.ds(s * rpt, rpt)])

        @pl.when(s == _NS - 1)
        def _():
            pltpu.sync_copy(z_hbm.at[pl.ds(0, last)],
                            acc.at[pl.ds(s * rpt, last)])

        plsc.subcore_barrier()

        def body(j, carry):
            pltpu.sync_copy(ones_v, acc.at[didx.at[j]], add=True)
            return carry

        lax.fori_loop(0, jpt, body, 0)
        plsc.subcore_barrier()

        @pl.when(s < _NS - 1)
        def _():
            pltpu.sync_copy(acc.at[pl.ds(s * rpt, rpt)],
                            out_hbm.at[c, pl.ds(s * rpt, rpt)])

        @pl.when(s == _NS - 1)
        def _():
            pltpu.sync_copy(acc.at[pl.ds(s * rpt, last)],
                            out_hbm.at[c, pl.ds(s * rpt, last)])

    return kern(dst2, zeros16, ones_hbm)


def _sc_scatter(m, src2, dst2, zeros32, n_acc, jpt):
    rpt, last = _row_partition(n_acc)
    mesh = plsc.VectorSubcoreMesh(core_axis_name="c", subcore_axis_name="s")
    jb = next(cand for cand in range(min(32, jpt), 1, -1)
              if jpt % cand == 0 and cand % 2 == 0)
    half = jb // 2

    @functools.partial(
        pl.kernel,
        out_type=jax.ShapeDtypeStruct((_NC, n_acc, 32), jnp.float32),
        mesh=mesh,
        compiler_params=pltpu.CompilerParams(use_tc_tiling_on_sc=False),
        scratch_types=[
            pltpu.VMEM_SHARED((n_acc, 32), jnp.float32),
            pltpu.VMEM((jb, 128), jnp.int32),
            pltpu.VMEM((jb, 128), jnp.int32),
            pltpu.VMEM((128, 32), jnp.float32),
            pltpu.VMEM((128, 32), jnp.float32),
            pltpu.SemaphoreType.DMA,
            pltpu.SemaphoreType.DMA,
        ],
    )
    def kern(m_hbm, src_hbm, dst_hbm, z_hbm, out_hbm,
             acc, sidx, didx, ra, rb, sa, sb):
        c = lax.axis_index("c")
        s = lax.axis_index("s")
        w = c * _NS + s

        @pl.when(s < _NS - 1)
        def _():
            pltpu.sync_copy(z_hbm, acc.at[pl.ds(s * rpt, rpt)])

        @pl.when(s == _NS - 1)
        def _():
            pltpu.sync_copy(z_hbm.at[pl.ds(0, last)],
                            acc.at[pl.ds(s * rpt, last)])

        plsc.subcore_barrier()

        def block(bi, bcarry):
            row0 = w * jpt + bi * jb
            pltpu.sync_copy(src_hbm.at[pl.ds(row0, jb)], sidx)
            pltpu.sync_copy(dst_hbm.at[pl.ds(row0, jb)], didx)
            pltpu.async_copy(m_hbm.at[sidx.at[0]], ra, sa)

            def body(t, carry):
                j = 2 * t
                pltpu.async_copy(m_hbm.at[sidx.at[j + 1]], rb, sb)
                pltpu.make_async_copy(m_hbm.at[sidx.at[j]], ra, sa).wait()
                pltpu.sync_copy(ra, acc.at[didx.at[j]], add=True)

                @pl.when(t < half - 1)
                def _():
                    pltpu.async_copy(m_hbm.at[sidx.at[j + 2]], ra, sa)

                pltpu.make_async_copy(m_hbm.at[sidx.at[j + 1]], rb, sb).wait()
                pltpu.sync_copy(rb, acc.at[didx.at[j + 1]], add=True)
                return carry

            lax.fori_loop(0, half, body, 0)
            return bcarry

        lax.fori_loop(0, jpt // jb, block, 0)
        plsc.subcore_barrier()

        @pl.when(s < _NS - 1)
        def _():
            pltpu.sync_copy(acc.at[pl.ds(s * rpt, rpt)],
                            out_hbm.at[c, pl.ds(s * rpt, rpt)])

        @pl.when(s == _NS - 1)
        def _():
            pltpu.sync_copy(acc.at[pl.ds(s * rpt, last)],
                            out_hbm.at[c, pl.ds(s * rpt, last)])

    return kern(m, src2, dst2, zeros32)


def _sc_gather(xb, idx, n, b):
    bpw = b // _NW
    mesh = plsc.VectorSubcoreMesh(core_axis_name="c", subcore_axis_name="s")

    @functools.partial(
        pl.kernel,
        out_type=jax.ShapeDtypeStruct((b, 32), jnp.float32),
        mesh=mesh,
        compiler_params=pltpu.CompilerParams(use_tc_tiling_on_sc=False),
        scratch_types=[
            pltpu.VMEM((bpw,), jnp.int32),
            pltpu.VMEM((bpw, 32), jnp.float32),
            pltpu.SemaphoreType.DMA,
        ],
    )
    def kern(x_hbm, idx_hbm, out_hbm, idxv, rows, sem):
        c = lax.axis_index("c")
        s = lax.axis_index("s")
        base = (s * _NC + c) * bpw
        pltpu.sync_copy(idx_hbm.at[pl.ds(base, bpw)], idxv)
        pltpu.async_copy(x_hbm.at[idxv], rows, sem).wait()
        pltpu.sync_copy(rows, out_hbm.at[pl.ds(base, bpw)])

    return kern(xb, idx)


def kernel(node_features, params, edges, input_node_indices):
    n, d = node_features.shape
    e = edges.shape[1]
    inv_e = 1.0 / float(e)
    br = 2000

    ffns = {name: [_bn_params(l) for l in params[name]]
            for name in ("pre", "conv1_prep", "conv1_upd",
                         "conv2_prep", "conv2_upd", "post")}

    jpt = -(-e // (_NW * 128))
    jpt += jpt % 2
    e_tot = _NW * 128 * jpt
    n_acc = n + 8
    dst2 = jnp.concatenate(
        [edges[0], jnp.full((e_tot - e,), n, jnp.int32)]).reshape(-1, 128)
    src2 = jnp.concatenate(
        [edges[1], jnp.zeros((e_tot - e,), jnp.int32)]).reshape(-1, 128)
    rpt, _ = _row_partition(n_acc)
    zeros32 = jnp.zeros((rpt, 32), jnp.float32)
    zeros16 = jnp.zeros((rpt, 16), jnp.float32)
    ones16 = jnp.ones((128, 16), jnp.float32)

    cnts = _sc_counts(dst2, zeros16, ones16, n_acc, jpt)

    x, m1 = _tc1(node_features, ffns["pre"], ffns["conv1_prep"],
                 inv_e, n, br)
    sums1 = _sc_scatter(m1, src2, dst2, zeros32, n_acc, jpt)
    xa, m2 = _tc_combine(x, sums1, cnts, ffns["conv1_upd"],
                         ffns["conv2_prep"], inv_e, n, br)
    sums2 = _sc_scatter(m2, src2, dst2, zeros32, n_acc, jpt)
    xb = _tc_combine(xa, sums2, cnts, ffns["conv2_upd"],
                     None, inv_e, n, br)[0]
    emb = _sc_gather(xb, input_node_indices, n, input_node_indices.shape[0])
    return _tc4(emb, ffns["post"], params["Wl"], params["bl"].reshape(1, -1))


# speedup vs baseline: 13.2246x; 1.0705x over previous
import functools

import jax
import jax.numpy as jnp
from jax import lax
from jax.experimental import pallas as pl
from jax.experimental.pallas import tpu as pltpu
from jax.experimental.pallas import tpu_sc as plsc

_NC = 2
_NS = 16
_NW = _NC * _NS

_SQRT2 = 1.4142135623730951


def _gelu(x):
    return x * 0.5 * (1.0 + lax.erf(x / _SQRT2))


def _bn_params(layer):
    s = layer["gamma"] / jnp.sqrt(layer["mvar"] + 1e-3)
    t = layer["beta"] - layer["mmean"] * s
    return [s.reshape(1, -1), t.reshape(1, -1), layer["W"],
            layer["b"].reshape(1, -1)]


def _layer(x, s, t, w, b):
    return _gelu(jnp.dot(x * s[...] + t[...], w[...],
                         preferred_element_type=jnp.float32) + b[...])


def _wspecs(arrs):
    return [pl.BlockSpec(a.shape, lambda i, r=len(a.shape): (0,) * r)
            for a in arrs]


def _tc1(nf, pre, prep, inv_e, n, br):
    def body(nf_r, s1, t1, w1, b1, s2, t2, w2, b2,
             p1s, p1t, p1w, p1b, p2s, p2t, p2w, p2b, x_r, m_r):
        x = _layer(_layer(nf_r[...], s1, t1, w1, b1), s2, t2, w2, b2)
        x_r[...] = x
        p = _layer(_layer(x, p1s, p1t, p1w, p1b), p2s, p2t, p2w, p2b)
        m_r[...] = p * inv_e

    d = nf.shape[1]
    ws = pre[0] + pre[1] + prep[0] + prep[1]
    return pl.pallas_call(
        body,
        grid=(n // br,),
        in_specs=[pl.BlockSpec((br, d), lambda i: (i, 0))] + _wspecs(ws),
        out_specs=[pl.BlockSpec((br, 32), lambda i: (i, 0)),
                   pl.BlockSpec((br, 32), lambda i: (i, 0))],
        out_shape=[jax.ShapeDtypeStruct((n, 32), jnp.float32),
                   jax.ShapeDtypeStruct((n, 32), jnp.float32)],
        compiler_params=pltpu.CompilerParams(
            dimension_semantics=("parallel",)),
    )(nf, *ws)


def _tc_combine(x, sums, cnts, upd, prep, inv_e, n, br):
    with_prep = prep is not None

    def body(*refs):
        if with_prep:
            (x_r, s_r, c_r, u1s, u1t, u1w, u1b, u2s, u2t, u2w, u2b,
             p1s, p1t, p1w, p1b, p2s, p2t, p2w, p2b, xa_r, m_r) = refs
        else:
            (x_r, s_r, c_r, u1s, u1t, u1w, u1b, u2s, u2t, u2w, u2b,
             xa_r) = refs
        counts = jnp.maximum(c_r[0, :, 0:1] + c_r[1, :, 0:1], 1.0)
        agg = (s_r[0] + s_r[1]) / counts
        x = x_r[...]
        h = jnp.concatenate([x, agg], axis=1)
        u = _layer(_layer(h, u1s, u1t, u1w, u1b), u2s, u2t, u2w, u2b)
        nrm = jnp.sqrt(jnp.maximum(jnp.sum(u * u, axis=1, keepdims=True),
                                   1e-12))
        xa = u / nrm + x
        xa_r[...] = xa
        if with_prep:
            p = _layer(_layer(xa, p1s, p1t, p1w, p1b), p2s, p2t, p2w, p2b)
            m_r[...] = p * inv_e

    ws = upd[0] + upd[1] + (prep[0] + prep[1] if with_prep else [])
    in_specs = [
        pl.BlockSpec((br, 32), lambda i: (i, 0)),
        pl.BlockSpec((2, br, 32), lambda i: (0, i, 0)),
        pl.BlockSpec((2, br, 16), lambda i: (0, i, 0)),
    ] + _wspecs(ws)
    nout = 2 if with_prep else 1
    out_specs = [pl.BlockSpec((br, 32), lambda i: (i, 0))] * nout
    out_shape = [jax.ShapeDtypeStruct((n, 32), jnp.float32)] * nout

    res = pl.pallas_call(
        body,
        grid=(n // br,),
        in_specs=in_specs,
        out_specs=out_specs,
        out_shape=out_shape,
        compiler_params=pltpu.CompilerParams(
            dimension_semantics=("parallel",)),
    )(x, sums, cnts, *ws)
    return res


def _tc4(emb, post, wl, bl):
    def body(e_r, q1s, q1t, q1w, q1b, q2s, q2t, q2w, q2b, wlr, blr, o_r):
        p = _layer(_layer(e_r[...], q1s, q1t, q1w, q1b), q2s, q2t, q2w, q2b)
        o_r[...] = jnp.dot(p, wlr[...],
                           preferred_element_type=jnp.float32) + blr[...]

    b = emb.shape[0]
    return pl.pallas_call(
        body,
        out_shape=jax.ShapeDtypeStruct((b, wl.shape[1]), jnp.float32),
    )(emb, *post[0], *post[1], wl, bl)


def _row_partition(n):
    rpt = ((n // _NS + 7) // 8) * 8
    last = n - (_NS - 1) * rpt
    assert last > 0 and last % 8 == 0 and rpt % 8 == 0
    return rpt, last


def _sc_counts(dst2, zeros16, ones_hbm, n_acc, jpt):
    rpt, last = _row_partition(n_acc)
    mesh = plsc.VectorSubcoreMesh(core_axis_name="c", subcore_axis_name="s")

    @functools.partial(
        pl.kernel,
        out_type=jax.ShapeDtypeStruct((_NC, n_acc, 16), jnp.float32),
        mesh=mesh,
        compiler_params=pltpu.CompilerParams(use_tc_tiling_on_sc=False),
        scratch_types=[
            pltpu.VMEM_SHARED((n_acc, 16), jnp.float32),
            pltpu.VMEM((jpt, 128), jnp.int32),
            pltpu.VMEM((128, 16), jnp.float32),
        ],
    )
    def kern(dst_hbm, z_hbm, o_hbm, out_hbm, acc, didx, ones_v):
        c = lax.axis_index("c")
        s = lax.axis_index("s")
        w = c * _NS + s
        pltpu.sync_copy(o_hbm, ones_v)
        pltpu.sync_copy(dst_hbm.at[pl.ds(w * jpt, jpt)], didx)

        @pl.when(s < _NS - 1)
        def _():
            pltpu.sync_copy(z_hbm, acc.at[pl.ds(s * rpt, rpt)])

        @pl.when(s == _NS - 1)
        def _():
            pltpu.sync_copy(z_hbm.at[pl.ds(0, last)],
                            acc.at[pl.ds(s * rpt, last)])

        plsc.subcore_barrier()

        def body(j, carry):
            pltpu.sync_copy(ones_v, acc.at[didx.at[j]], add=True)
            return carry

        lax.fori_loop(0, jpt, body, 0)
        plsc.subcore_barrier()

        @pl.when(s < _NS - 1)
        def _():
            pltpu.sync_copy(acc.at[pl.ds(s * rpt, rpt)],
                            out_hbm.at[c, pl.ds(s * rpt, rpt)])

        @pl.when(s == _NS - 1)
        def _():
            pltpu.sync_copy(acc.at[pl.ds(s * rpt, last)],
                            out_hbm.at[c, pl.ds(s * rpt, last)])

    return kern(dst2, zeros16, ones_hbm)


def _sc_scatter(m, src2, dst2, zeros32, n_acc, jpt):
    rpt, last = _row_partition(n_acc)
    mesh = plsc.VectorSubcoreMesh(core_axis_name="c", subcore_axis_name="s")
    jb = next(cand for cand in range(min(32, jpt), 3, -1)
              if jpt % cand == 0 and cand % 4 == 0)

    @functools.partial(
        pl.kernel,
        out_type=jax.ShapeDtypeStruct((_NC, n_acc, 32), jnp.float32),
        mesh=mesh,
        compiler_params=pltpu.CompilerParams(use_tc_tiling_on_sc=False),
        scratch_types=[
            pltpu.VMEM_SHARED((n_acc, 32), jnp.float32),
            pltpu.VMEM((jb, 128), jnp.int32),
            pltpu.VMEM((jb, 128), jnp.int32),
            pltpu.VMEM((128, 32), jnp.float32),
            pltpu.VMEM((128, 32), jnp.float32),
            pltpu.VMEM((128, 32), jnp.float32),
            pltpu.VMEM((128, 32), jnp.float32),
            pltpu.SemaphoreType.DMA,
            pltpu.SemaphoreType.DMA,
            pltpu.SemaphoreType.DMA,
            pltpu.SemaphoreType.DMA,
        ],
    )
    def kern(m_hbm, src_hbm, dst_hbm, z_hbm, out_hbm,
             acc, sidx, didx, r0, r1, r2, r3, s0, s1, s2, s3):
        bufs = (r0, r1, r2, r3)
        sems = (s0, s1, s2, s3)
        c = lax.axis_index("c")
        s = lax.axis_index("s")
        w = c * _NS + s

        @pl.when(s < _NS - 1)
        def _():
            pltpu.sync_copy(z_hbm, acc.at[pl.ds(s * rpt, rpt)])

        @pl.when(s == _NS - 1)
        def _():
            pltpu.sync_copy(z_hbm.at[pl.ds(0, last)],
                            acc.at[pl.ds(s * rpt, last)])

        plsc.subcore_barrier()

        def block(bi, bcarry):
            row0 = w * jpt + bi * jb
            pltpu.sync_copy(src_hbm.at[pl.ds(row0, jb)], sidx)
            pltpu.sync_copy(dst_hbm.at[pl.ds(row0, jb)], didx)
            for l in range(3):
                pltpu.async_copy(m_hbm.at[sidx.at[l]], bufs[l], sems[l])

            def quad(t, carry):
                for l in range(4):
                    j = 4 * t + l
                    pltpu.make_async_copy(m_hbm.at[sidx.at[j]],
                                          bufs[l], sems[l]).wait()
                    pltpu.sync_copy(bufs[l], acc.at[didx.at[j]], add=True)

                    @pl.when(j + 3 < jb)
                    def _():
                        pltpu.async_copy(m_hbm.at[sidx.at[j + 3]],
                                         bufs[(l + 3) % 4], sems[(l + 3) % 4])
                return carry

            lax.fori_loop(0, jb // 4, quad, 0)
            return bcarry

        lax.fori_loop(0, jpt // jb, block, 0)
        plsc.subcore_barrier()

        @pl.when(s < _NS - 1)
        def _():
            pltpu.sync_copy(acc.at[pl.ds(s * rpt, rpt)],
                            out_hbm.at[c, pl.ds(s * rpt, rpt)])

        @pl.when(s == _NS - 1)
        def _():
            pltpu.sync_copy(acc.at[pl.ds(s * rpt, last)],
                            out_hbm.at[c, pl.ds(s * rpt, last)])

    return kern(m, src2, dst2, zeros32)


def _sc_gather(xb, idx, n, b):
    bpw = b // _NW
    mesh = plsc.VectorSubcoreMesh(core_axis_name="c", subcore_axis_name="s")

    @functools.partial(
        pl.kernel,
        out_type=jax.ShapeDtypeStruct((b, 32), jnp.float32),
        mesh=mesh,
        compiler_params=pltpu.CompilerParams(use_tc_tiling_on_sc=False),
        scratch_types=[
            pltpu.VMEM((bpw,), jnp.int32),
            pltpu.VMEM((bpw, 32), jnp.float32),
            pltpu.SemaphoreType.DMA,
        ],
    )
    def kern(x_hbm, idx_hbm, out_hbm, idxv, rows, sem):
        c = lax.axis_index("c")
        s = lax.axis_index("s")
        base = (s * _NC + c) * bpw
        pltpu.sync_copy(idx_hbm.at[pl.ds(base, bpw)], idxv)
        pltpu.async_copy(x_hbm.at[idxv], rows, sem).wait()
        pltpu.sync_copy(rows, out_hbm.at[pl.ds(base, bpw)])

    return kern(xb, idx)


def kernel(node_features, params, edges, input_node_indices):
    n, d = node_features.shape
    e = edges.shape[1]
    inv_e = 1.0 / float(e)
    br = 2000

    ffns = {name: [_bn_params(l) for l in params[name]]
            for name in ("pre", "conv1_prep", "conv1_upd",
                         "conv2_prep", "conv2_upd", "post")}

    jpt = -(-e // (_NW * 128))
    jpt += jpt % 2
    e_tot = _NW * 128 * jpt
    n_acc = n + 8
    dst2 = jnp.concatenate(
        [edges[0], jnp.full((e_tot - e,), n, jnp.int32)]).reshape(-1, 128)
    src2 = jnp.concatenate(
        [edges[1], jnp.zeros((e_tot - e,), jnp.int32)]).reshape(-1, 128)
    rpt, _ = _row_partition(n_acc)
    zeros32 = jnp.zeros((rpt, 32), jnp.float32)
    zeros16 = jnp.zeros((rpt, 16), jnp.float32)
    ones16 = jnp.ones((128, 16), jnp.float32)

    cnts = _sc_counts(dst2, zeros16, ones16, n_acc, jpt)

    x, m1 = _tc1(node_features, ffns["pre"], ffns["conv1_prep"],
                 inv_e, n, br)
    sums1 = _sc_scatter(m1, src2, dst2, zeros32, n_acc, jpt)
    xa, m2 = _tc_combine(x, sums1, cnts, ffns["conv1_upd"],
                         ffns["conv2_prep"], inv_e, n, br)
    sums2 = _sc_scatter(m2, src2, dst2, zeros32, n_acc, jpt)
    xb = _tc_combine(xa, sums2, cnts, ffns["conv2_upd"],
                     None, inv_e, n, br)[0]
    emb = _sc_gather(xb, input_node_indices, n, input_node_indices.shape[0])
    return _tc4(emb, ffns["post"], params["Wl"], params["bl"].reshape(1, -1))


# speedup vs baseline: 13.4450x; 1.0167x over previous
import functools

import jax
import jax.numpy as jnp
from jax import lax
from jax.experimental import pallas as pl
from jax.experimental.pallas import tpu as pltpu
from jax.experimental.pallas import tpu_sc as plsc

_NC = 2
_NS = 16
_NW = _NC * _NS

_SQRT2 = 1.4142135623730951


def _gelu(x):
    return x * 0.5 * (1.0 + lax.erf(x / _SQRT2))


def _bn_params(layer):
    s = layer["gamma"] / jnp.sqrt(layer["mvar"] + 1e-3)
    t = layer["beta"] - layer["mmean"] * s
    return [s.reshape(1, -1), t.reshape(1, -1), layer["W"],
            layer["b"].reshape(1, -1)]


def _layer(x, s, t, w, b):
    return _gelu(jnp.dot(x * s[...] + t[...], w[...],
                         preferred_element_type=jnp.float32) + b[...])


def _wspecs(arrs):
    return [pl.BlockSpec(a.shape, lambda i, r=len(a.shape): (0,) * r)
            for a in arrs]


def _tc1(nf, pre, prep, inv_e, n, br):
    def body(nf_r, s1, t1, w1, b1, s2, t2, w2, b2,
             p1s, p1t, p1w, p1b, p2s, p2t, p2w, p2b, x_r, m_r):
        x = _layer(_layer(nf_r[...], s1, t1, w1, b1), s2, t2, w2, b2)
        x_r[...] = x
        p = _layer(_layer(x, p1s, p1t, p1w, p1b), p2s, p2t, p2w, p2b)
        m_r[...] = p * inv_e

    d = nf.shape[1]
    ws = pre[0] + pre[1] + prep[0] + prep[1]
    return pl.pallas_call(
        body,
        grid=(n // br,),
        in_specs=[pl.BlockSpec((br, d), lambda i: (i, 0))] + _wspecs(ws),
        out_specs=[pl.BlockSpec((br, 32), lambda i: (i, 0)),
                   pl.BlockSpec((br, 32), lambda i: (i, 0))],
        out_shape=[jax.ShapeDtypeStruct((n, 32), jnp.float32),
                   jax.ShapeDtypeStruct((n, 32), jnp.float32)],
        compiler_params=pltpu.CompilerParams(
            dimension_semantics=("parallel",)),
    )(nf, *ws)


def _tc_combine(x, sums, cnts, upd, prep, final, inv_e, n, br):
    with_prep = prep is not None

    def body(*refs):
        if with_prep:
            (x_r, s_r, c_r, u1s, u1t, u1w, u1b, u2s, u2t, u2w, u2b,
             p1s, p1t, p1w, p1b, p2s, p2t, p2w, p2b, xa_r, m_r) = refs
        else:
            (x_r, s_r, c_r, u1s, u1t, u1w, u1b, u2s, u2t, u2w, u2b,
             q1s, q1t, q1w, q1b, q2s, q2t, q2w, q2b, wlr, blr, y_r) = refs
        counts = jnp.maximum(c_r[0, :, 0:1] + c_r[1, :, 0:1], 1.0)
        agg = (s_r[0] + s_r[1]) / counts
        x = x_r[...]
        h = jnp.concatenate([x, agg], axis=1)
        u = _layer(_layer(h, u1s, u1t, u1w, u1b), u2s, u2t, u2w, u2b)
        nrm = jnp.sqrt(jnp.maximum(jnp.sum(u * u, axis=1, keepdims=True),
                                   1e-12))
        xa = u / nrm + x
        if with_prep:
            xa_r[...] = xa
            p = _layer(_layer(xa, p1s, p1t, p1w, p1b), p2s, p2t, p2w, p2b)
            m_r[...] = p * inv_e
        else:
            p = _layer(_layer(xa, q1s, q1t, q1w, q1b), q2s, q2t, q2w, q2b)
            y_r[...] = jnp.dot(p, wlr[...],
                               preferred_element_type=jnp.float32) + blr[...]

    if with_prep:
        ws = upd[0] + upd[1] + prep[0] + prep[1]
        ncls = None
    else:
        post, wl, bl = final
        ws = upd[0] + upd[1] + post[0] + post[1] + [wl, bl]
        ncls = wl.shape[1]
    in_specs = [
        pl.BlockSpec((br, 32), lambda i: (i, 0)),
        pl.BlockSpec((2, br, 32), lambda i: (0, i, 0)),
        pl.BlockSpec((2, br, 16), lambda i: (0, i, 0)),
    ] + _wspecs(ws)
    if with_prep:
        out_specs = [pl.BlockSpec((br, 32), lambda i: (i, 0))] * 2
        out_shape = [jax.ShapeDtypeStruct((n, 32), jnp.float32)] * 2
    else:
        out_specs = [pl.BlockSpec((br, ncls), lambda i: (i, 0))]
        out_shape = [jax.ShapeDtypeStruct((n, ncls), jnp.float32)]

    res = pl.pallas_call(
        body,
        grid=(n // br,),
        in_specs=in_specs,
        out_specs=out_specs,
        out_shape=out_shape,
        compiler_params=pltpu.CompilerParams(
            dimension_semantics=("parallel",)),
    )(x, sums, cnts, *ws)
    return res


def _row_partition(n):
    rpt = ((n // _NS + 7) // 8) * 8
    last = n - (_NS - 1) * rpt
    assert last > 0 and last % 8 == 0 and rpt % 8 == 0
    return rpt, last


def _sc_counts(dst2, zeros16, ones_hbm, n_acc, jpt):
    rpt, last = _row_partition(n_acc)
    mesh = plsc.VectorSubcoreMesh(core_axis_name="c", subcore_axis_name="s")

    @functools.partial(
        pl.kernel,
        out_type=jax.ShapeDtypeStruct((_NC, n_acc, 16), jnp.float32),
        mesh=mesh,
        compiler_params=pltpu.CompilerParams(use_tc_tiling_on_sc=False),
        scratch_types=[
            pltpu.VMEM_SHARED((n_acc, 16), jnp.float32),
            pltpu.VMEM((jpt, 128), jnp.int32),
            pltpu.VMEM((128, 16), jnp.float32),
        ],
    )
    def kern(dst_hbm, z_hbm, o_hbm, out_hbm, acc, didx, ones_v):
        c = lax.axis_index("c")
        s = lax.axis_index("s")
        w = c * _NS + s
        pltpu.sync_copy(o_hbm, ones_v)
        pltpu.sync_copy(dst_hbm.at[pl.ds(w * jpt, jpt)], didx)

        @pl.when(s < _NS - 1)
        def _():
            pltpu.sync_copy(z_hbm, acc.at[pl.ds(s * rpt, rpt)])

        @pl.when(s == _NS - 1)
        def _():
            pltpu.sync_copy(z_hbm.at[pl.ds(0, last)],
                            acc.at[pl.ds(s * rpt, last)])

        plsc.subcore_barrier()

        def body(j, carry):
            pltpu.sync_copy(ones_v, acc.at[didx.at[j]], add=True)
            return carry

        lax.fori_loop(0, jpt, body, 0)
        plsc.subcore_barrier()

        @pl.when(s < _NS - 1)
        def _():
            pltpu.sync_copy(acc.at[pl.ds(s * rpt, rpt)],
                            out_hbm.at[c, pl.ds(s * rpt, rpt)])

        @pl.when(s == _NS - 1)
        def _():
            pltpu.sync_copy(acc.at[pl.ds(s * rpt, last)],
                            out_hbm.at[c, pl.ds(s * rpt, last)])

    return kern(dst2, zeros16, ones_hbm)


def _sc_scatter(m, src2, dst2, zeros32, n_acc, jpt):
    rpt, last = _row_partition(n_acc)
    mesh = plsc.VectorSubcoreMesh(core_axis_name="c", subcore_axis_name="s")
    jb = next(cand for cand in range(min(32, jpt), 3, -1)
              if jpt % cand == 0 and cand % 4 == 0)

    @functools.partial(
        pl.kernel,
        out_type=jax.ShapeDtypeStruct((_NC, n_acc, 32), jnp.float32),
        mesh=mesh,
        compiler_params=pltpu.CompilerParams(use_tc_tiling_on_sc=False),
        scratch_types=[
            pltpu.VMEM_SHARED((n_acc, 32), jnp.float32),
            pltpu.VMEM((jb, 128), jnp.int32),
            pltpu.VMEM((jb, 128), jnp.int32),
            pltpu.VMEM((128, 32), jnp.float32),
            pltpu.VMEM((128, 32), jnp.float32),
            pltpu.VMEM((128, 32), jnp.float32),
            pltpu.VMEM((128, 32), jnp.float32),
            pltpu.SemaphoreType.DMA,
            pltpu.SemaphoreType.DMA,
            pltpu.SemaphoreType.DMA,
            pltpu.SemaphoreType.DMA,
            pltpu.SemaphoreType.DMA,
            pltpu.SemaphoreType.DMA,
            pltpu.SemaphoreType.DMA,
            pltpu.SemaphoreType.DMA,
        ],
    )
    def kern(m_hbm, src_hbm, dst_hbm, z_hbm, out_hbm,
             acc, sidx, didx, r0, r1, r2, r3,
             g0, g1, g2, g3, t0, t1, t2, t3):
        bufs = (r0, r1, r2, r3)
        gsems = (g0, g1, g2, g3)
        ssems = (t0, t1, t2, t3)
        c = lax.axis_index("c")
        s = lax.axis_index("s")
        w = c * _NS + s

        @pl.when(s < _NS - 1)
        def _():
            pltpu.sync_copy(z_hbm, acc.at[pl.ds(s * rpt, rpt)])

        @pl.when(s == _NS - 1)
        def _():
            pltpu.sync_copy(z_hbm.at[pl.ds(0, last)],
                            acc.at[pl.ds(s * rpt, last)])

        plsc.subcore_barrier()

        def block(bi, bcarry):
            row0 = w * jpt + bi * jb
            pltpu.sync_copy(src_hbm.at[pl.ds(row0, jb)], sidx)
            pltpu.sync_copy(dst_hbm.at[pl.ds(row0, jb)], didx)
            for l in range(3):
                pltpu.async_copy(m_hbm.at[sidx.at[l]], bufs[l], gsems[l])

            def quad(t, carry):
                for l in range(4):
                    j = 4 * t + l
                    pltpu.make_async_copy(m_hbm.at[sidx.at[j]],
                                          bufs[l], gsems[l]).wait()
                    pltpu.async_copy(bufs[l], acc.at[didx.at[j]],
                                     ssems[l], add=True)

                    @pl.when(j + 3 < jb)
                    def _():
                        ln = (l + 3) % 4

                        @pl.when(j > 0)
                        def _():
                            pltpu.make_async_copy(
                                bufs[ln], acc.at[didx.at[j]],
                                ssems[ln]).wait()

                        pltpu.async_copy(m_hbm.at[sidx.at[j + 3]],
                                         bufs[ln], gsems[ln])
                return carry

            lax.fori_loop(0, jb // 4, quad, 0)
            for l in range(4):
                pltpu.make_async_copy(bufs[l], acc.at[didx.at[l]],
                                      ssems[l]).wait()
            return bcarry

        lax.fori_loop(0, jpt // jb, block, 0)
        plsc.subcore_barrier()

        @pl.when(s < _NS - 1)
        def _():
            pltpu.sync_copy(acc.at[pl.ds(s * rpt, rpt)],
                            out_hbm.at[c, pl.ds(s * rpt, rpt)])

        @pl.when(s == _NS - 1)
        def _():
            pltpu.sync_copy(acc.at[pl.ds(s * rpt, last)],
                            out_hbm.at[c, pl.ds(s * rpt, last)])

    return kern(m, src2, dst2, zeros32)


def _sc_gather(xb, idx, b, width):
    bpw = b // _NW
    mesh = plsc.VectorSubcoreMesh(core_axis_name="c", subcore_axis_name="s")

    @functools.partial(
        pl.kernel,
        out_type=jax.ShapeDtypeStruct((b, width), jnp.float32),
        mesh=mesh,
        compiler_params=pltpu.CompilerParams(use_tc_tiling_on_sc=False),
        scratch_types=[
            pltpu.VMEM((bpw,), jnp.int32),
            pltpu.VMEM((bpw, width), jnp.float32),
            pltpu.SemaphoreType.DMA,
        ],
    )
    def kern(x_hbm, idx_hbm, out_hbm, idxv, rows, sem):
        c = lax.axis_index("c")
        s = lax.axis_index("s")
        base = (s * _NC + c) * bpw
        pltpu.sync_copy(idx_hbm.at[pl.ds(base, bpw)], idxv)
        pltpu.async_copy(x_hbm.at[idxv], rows, sem).wait()
        pltpu.sync_copy(rows, out_hbm.at[pl.ds(base, bpw)])

    return kern(xb, idx)


def kernel(node_features, params, edges, input_node_indices):
    n, d = node_features.shape
    e = edges.shape[1]
    inv_e = 1.0 / float(e)
    br = 2000

    ffns = {name: [_bn_params(l) for l in params[name]]
            for name in ("pre", "conv1_prep", "conv1_upd",
                         "conv2_prep", "conv2_upd", "post")}

    jpt = -(-e // (_NW * 128))
    jpt += jpt % 2
    e_tot = _NW * 128 * jpt
    n_acc = n + 8
    dst2 = jnp.concatenate(
        [edges[0], jnp.full((e_tot - e,), n, jnp.int32)]).reshape(-1, 128)
    src2 = jnp.concatenate(
        [edges[1], jnp.zeros((e_tot - e,), jnp.int32)]).reshape(-1, 128)
    rpt, _ = _row_partition(n_acc)
    zeros32 = jnp.zeros((rpt, 32), jnp.float32)
    zeros16 = jnp.zeros((rpt, 16), jnp.float32)
    ones16 = jnp.ones((128, 16), jnp.float32)

    cnts = _sc_counts(dst2, zeros16, ones16, n_acc, jpt)

    x, m1 = _tc1(node_features, ffns["pre"], ffns["conv1_prep"],
                 inv_e, n, br)
    sums1 = _sc_scatter(m1, src2, dst2, zeros32, n_acc, jpt)
    xa, m2 = _tc_combine(x, sums1, cnts, ffns["conv1_upd"],
                         ffns["conv2_prep"], None, inv_e, n, br)
    sums2 = _sc_scatter(m2, src2, dst2, zeros32, n_acc, jpt)
    ncls = params["Wl"].shape[1]
    y_all = _tc_combine(xa, sums2, cnts, ffns["conv2_upd"], None,
                        (ffns["post"], params["Wl"],
                         params["bl"].reshape(1, -1)),
                        inv_e, n, br)[0]
    return _sc_gather(y_all, input_node_indices,
                      input_node_indices.shape[0], ncls)


# speedup vs baseline: 16.0340x; 1.1926x over previous
import functools

import jax
import jax.numpy as jnp
from jax import lax
from jax.experimental import pallas as pl
from jax.experimental.pallas import tpu as pltpu
from jax.experimental.pallas import tpu_sc as plsc

_NC = 2
_NS = 16
_NW = _NC * _NS

_SQRT2 = 1.4142135623730951


def _gelu(x):
    return x * 0.5 * (1.0 + lax.erf(x / _SQRT2))


def _bn_params(layer):
    s = layer["gamma"] / jnp.sqrt(layer["mvar"] + 1e-3)
    t = layer["beta"] - layer["mmean"] * s
    return [s.reshape(1, -1), t.reshape(1, -1), layer["W"],
            layer["b"].reshape(1, -1)]


def _layer(x, s, t, w, b):
    return _gelu(jnp.dot(x * s[...] + t[...], w[...],
                         preferred_element_type=jnp.float32) + b[...])


def _wspecs(arrs):
    return [pl.BlockSpec(a.shape, lambda i, r=len(a.shape): (0,) * r)
            for a in arrs]


def _tc1(nf, pre, prep, inv_e, n, br):
    def body(nf_r, s1, t1, w1, b1, s2, t2, w2, b2,
             p1s, p1t, p1w, p1b, p2s, p2t, p2w, p2b, x_r, m_r):
        x = _layer(_layer(nf_r[...], s1, t1, w1, b1), s2, t2, w2, b2)
        x_r[...] = x
        p = _layer(_layer(x, p1s, p1t, p1w, p1b), p2s, p2t, p2w, p2b)
        m_r[...] = (p * inv_e).astype(jnp.bfloat16)

    d = nf.shape[1]
    ws = pre[0] + pre[1] + prep[0] + prep[1]
    return pl.pallas_call(
        body,
        grid=(n // br,),
        in_specs=[pl.BlockSpec((br, d), lambda i: (i, 0))] + _wspecs(ws),
        out_specs=[pl.BlockSpec((br, 32), lambda i: (i, 0)),
                   pl.BlockSpec((br, 32), lambda i: (i, 0))],
        out_shape=[jax.ShapeDtypeStruct((n, 32), jnp.float32),
                   jax.ShapeDtypeStruct((n, 32), jnp.bfloat16)],
        compiler_params=pltpu.CompilerParams(
            dimension_semantics=("parallel",)),
    )(nf, *ws)


def _tc_combine(x, sums, cnts, upd, prep, final, inv_e, n, br):
    with_prep = prep is not None

    def body(*refs):
        if with_prep:
            (x_r, s_r, c_r, u1s, u1t, u1w, u1b, u2s, u2t, u2w, u2b,
             p1s, p1t, p1w, p1b, p2s, p2t, p2w, p2b, xa_r, m_r) = refs
        else:
            (x_r, s_r, c_r, u1s, u1t, u1w, u1b, u2s, u2t, u2w, u2b,
             q1s, q1t, q1w, q1b, q2s, q2t, q2w, q2b, wlr, blr, y_r) = refs
        counts = jnp.maximum(c_r[0, :, 0:1] + c_r[1, :, 0:1], 1.0)
        agg = (s_r[0].astype(jnp.float32)
               + s_r[1].astype(jnp.float32)) / counts
        x = x_r[...]
        h = jnp.concatenate([x, agg], axis=1)
        u = _layer(_layer(h, u1s, u1t, u1w, u1b), u2s, u2t, u2w, u2b)
        nrm = jnp.sqrt(jnp.maximum(jnp.sum(u * u, axis=1, keepdims=True),
                                   1e-12))
        xa = u / nrm + x
        if with_prep:
            xa_r[...] = xa
            p = _layer(_layer(xa, p1s, p1t, p1w, p1b), p2s, p2t, p2w, p2b)
            m_r[...] = (p * inv_e).astype(jnp.bfloat16)
        else:
            p = _layer(_layer(xa, q1s, q1t, q1w, q1b), q2s, q2t, q2w, q2b)
            y_r[...] = jnp.dot(p, wlr[...],
                               preferred_element_type=jnp.float32) + blr[...]

    if with_prep:
        ws = upd[0] + upd[1] + prep[0] + prep[1]
        ncls = None
    else:
        post, wl, bl = final
        ws = upd[0] + upd[1] + post[0] + post[1] + [wl, bl]
        ncls = wl.shape[1]
    in_specs = [
        pl.BlockSpec((br, 32), lambda i: (i, 0)),
        pl.BlockSpec((2, br, 32), lambda i: (0, i, 0)),
        pl.BlockSpec((2, br, 16), lambda i: (0, i, 0)),
    ] + _wspecs(ws)
    if with_prep:
        out_specs = [pl.BlockSpec((br, 32), lambda i: (i, 0))] * 2
        out_shape = [jax.ShapeDtypeStruct((n, 32), jnp.float32),
                     jax.ShapeDtypeStruct((n, 32), jnp.bfloat16)]
    else:
        out_specs = [pl.BlockSpec((br, ncls), lambda i: (i, 0))]
        out_shape = [jax.ShapeDtypeStruct((n, ncls), jnp.float32)]

    res = pl.pallas_call(
        body,
        grid=(n // br,),
        in_specs=in_specs,
        out_specs=out_specs,
        out_shape=out_shape,
        compiler_params=pltpu.CompilerParams(
            dimension_semantics=("parallel",)),
    )(x, sums, cnts, *ws)
    return res


def _row_partition(n):
    rpt = ((n // _NS + 7) // 8) * 8
    last = n - (_NS - 1) * rpt
    assert last > 0 and last % 8 == 0 and rpt % 8 == 0
    return rpt, last


def _sc_counts(dst2, zeros16, ones_hbm, n_acc, jpt):
    rpt, last = _row_partition(n_acc)
    mesh = plsc.VectorSubcoreMesh(core_axis_name="c", subcore_axis_name="s")

    @functools.partial(
        pl.kernel,
        out_type=jax.ShapeDtypeStruct((_NC, n_acc, 16), jnp.float32),
        mesh=mesh,
        compiler_params=pltpu.CompilerParams(use_tc_tiling_on_sc=False),
        scratch_types=[
            pltpu.VMEM_SHARED((n_acc, 16), jnp.float32),
            pltpu.VMEM((jpt, 128), jnp.int32),
            pltpu.VMEM((128, 16), jnp.float32),
        ],
    )
    def kern(dst_hbm, z_hbm, o_hbm, out_hbm, acc, didx, ones_v):
        c = lax.axis_index("c")
        s = lax.axis_index("s")
        w = c * _NS + s
        pltpu.sync_copy(o_hbm, ones_v)
        pltpu.sync_copy(dst_hbm.at[pl.ds(w * jpt, jpt)], didx)

        @pl.when(s < _NS - 1)
        def _():
            pltpu.sync_copy(z_hbm, acc.at[pl.ds(s * rpt, rpt)])

        @pl.when(s == _NS - 1)
        def _():
            pltpu.sync_copy(z_hbm.at[pl.ds(0, last)],
                            acc.at[pl.ds(s * rpt, last)])

        plsc.subcore_barrier()

        def body(j, carry):
            pltpu.sync_copy(ones_v, acc.at[didx.at[j]], add=True)
            return carry

        lax.fori_loop(0, jpt, body, 0)
        plsc.subcore_barrier()

        @pl.when(s < _NS - 1)
        def _():
            pltpu.sync_copy(acc.at[pl.ds(s * rpt, rpt)],
                            out_hbm.at[c, pl.ds(s * rpt, rpt)])

        @pl.when(s == _NS - 1)
        def _():
            pltpu.sync_copy(acc.at[pl.ds(s * rpt, last)],
                            out_hbm.at[c, pl.ds(s * rpt, last)])

    return kern(dst2, zeros16, ones_hbm)


def _sc_scatter(m, src2, dst2, zeros32, n_acc, jpt):
    n = m.shape[0]
    rpt, last = _row_partition(n_acc)
    rptm, lastm = _row_partition(n)
    mesh = plsc.VectorSubcoreMesh(core_axis_name="c", subcore_axis_name="s")
    jb = next(cand for cand in range(min(32, jpt), 3, -1)
              if jpt % cand == 0 and cand % 4 == 0)

    @functools.partial(
        pl.kernel,
        out_type=jax.ShapeDtypeStruct((_NC, n_acc, 32), jnp.bfloat16),
        mesh=mesh,
        compiler_params=pltpu.CompilerParams(use_tc_tiling_on_sc=False),
        scratch_types=[
            pltpu.VMEM_SHARED((n_acc, 32), jnp.bfloat16),
            pltpu.VMEM_SHARED((n, 32), jnp.bfloat16),
            pltpu.VMEM((jb, 128), jnp.int32),
            pltpu.VMEM((jb, 128), jnp.int32),
            pltpu.VMEM((128, 32), jnp.bfloat16),
            pltpu.VMEM((128, 32), jnp.bfloat16),
            pltpu.VMEM((128, 32), jnp.bfloat16),
            pltpu.VMEM((128, 32), jnp.bfloat16),
            pltpu.SemaphoreType.DMA,
            pltpu.SemaphoreType.DMA,
            pltpu.SemaphoreType.DMA,
            pltpu.SemaphoreType.DMA,
            pltpu.SemaphoreType.DMA,
            pltpu.SemaphoreType.DMA,
            pltpu.SemaphoreType.DMA,
            pltpu.SemaphoreType.DMA,
        ],
    )
    def kern(m_hbm, src_hbm, dst_hbm, z_hbm, out_hbm,
             acc, mb, sidx, didx, r0, r1, r2, r3,
             g0, g1, g2, g3, t0, t1, t2, t3):
        bufs = (r0, r1, r2, r3)
        gsems = (g0, g1, g2, g3)
        ssems = (t0, t1, t2, t3)
        c = lax.axis_index("c")
        s = lax.axis_index("s")
        w = c * _NS + s

        @pl.when(s < _NS - 1)
        def _():
            pltpu.sync_copy(z_hbm, acc.at[pl.ds(s * rpt, rpt)])
            pltpu.sync_copy(m_hbm.at[pl.ds(s * rptm, rptm)],
                            mb.at[pl.ds(s * rptm, rptm)])

        @pl.when(s == _NS - 1)
        def _():
            pltpu.sync_copy(z_hbm.at[pl.ds(0, last)],
                            acc.at[pl.ds(s * rpt, last)])
            pltpu.sync_copy(m_hbm.at[pl.ds(s * rptm, lastm)],
                            mb.at[pl.ds(s * rptm, lastm)])

        plsc.subcore_barrier()

        def block(bi, bcarry):
            row0 = w * jpt + bi * jb
            pltpu.sync_copy(src_hbm.at[pl.ds(row0, jb)], sidx)
            pltpu.sync_copy(dst_hbm.at[pl.ds(row0, jb)], didx)
            for l in range(3):
                pltpu.async_copy(mb.at[sidx.at[l]], bufs[l], gsems[l])

            def quad(t, carry):
                for l in range(4):
                    j = 4 * t + l
                    pltpu.make_async_copy(mb.at[sidx.at[j]],
                                          bufs[l], gsems[l]).wait()
                    pltpu.async_copy(bufs[l], acc.at[didx.at[j]],
                                     ssems[l], add=True)

                    @pl.when(j + 3 < jb)
                    def _():
                        ln = (l + 3) % 4

                        @pl.when(j > 0)
                        def _():
                            pltpu.make_async_copy(
                                bufs[ln], acc.at[didx.at[j]],
                                ssems[ln]).wait()

                        pltpu.async_copy(mb.at[sidx.at[j + 3]],
                                         bufs[ln], gsems[ln])
                return carry

            lax.fori_loop(0, jb // 4, quad, 0)
            for l in range(4):
                pltpu.make_async_copy(bufs[l], acc.at[didx.at[l]],
                                      ssems[l]).wait()
            return bcarry

        lax.fori_loop(0, jpt // jb, block, 0)
        plsc.subcore_barrier()

        @pl.when(s < _NS - 1)
        def _():
            pltpu.sync_copy(acc.at[pl.ds(s * rpt, rpt)],
                            out_hbm.at[c, pl.ds(s * rpt, rpt)])

        @pl.when(s == _NS - 1)
        def _():
            pltpu.sync_copy(acc.at[pl.ds(s * rpt, last)],
                            out_hbm.at[c, pl.ds(s * rpt, last)])

    return kern(m, src2, dst2, zeros32)


def _sc_gather(xb, idx, b, width):
    bpw = b // _NW
    mesh = plsc.VectorSubcoreMesh(core_axis_name="c", subcore_axis_name="s")

    @functools.partial(
        pl.kernel,
        out_type=jax.ShapeDtypeStruct((b, width), jnp.float32),
        mesh=mesh,
        compiler_params=pltpu.CompilerParams(use_tc_tiling_on_sc=False),
        scratch_types=[
            pltpu.VMEM((bpw,), jnp.int32),
            pltpu.VMEM((bpw, width), jnp.float32),
            pltpu.SemaphoreType.DMA,
        ],
    )
    def kern(x_hbm, idx_hbm, out_hbm, idxv, rows, sem):
        c = lax.axis_index("c")
        s = lax.axis_index("s")
        base = (s * _NC + c) * bpw
        pltpu.sync_copy(idx_hbm.at[pl.ds(base, bpw)], idxv)
        pltpu.async_copy(x_hbm.at[idxv], rows, sem).wait()
        pltpu.sync_copy(rows, out_hbm.at[pl.ds(base, bpw)])

    return kern(xb, idx)


def kernel(node_features, params, edges, input_node_indices):
    n, d = node_features.shape
    e = edges.shape[1]
    inv_e = 1.0 / float(e)
    br = 2000

    ffns = {name: [_bn_params(l) for l in params[name]]
            for name in ("pre", "conv1_prep", "conv1_upd",
                         "conv2_prep", "conv2_upd", "post")}

    jpt = -(-e // (_NW * 128))
    jpt += jpt % 2
    e_tot = _NW * 128 * jpt
    n_acc = n + 8
    dst2 = jnp.concatenate(
        [edges[0], jnp.full((e_tot - e,), n, jnp.int32)]).reshape(-1, 128)
    src2 = jnp.concatenate(
        [edges[1], jnp.zeros((e_tot - e,), jnp.int32)]).reshape(-1, 128)
    rpt, _ = _row_partition(n_acc)
    zeros32 = jnp.zeros((rpt, 32), jnp.bfloat16)
    zeros16 = jnp.zeros((rpt, 16), jnp.float32)
    ones16 = jnp.ones((128, 16), jnp.float32)

    cnts = _sc_counts(dst2, zeros16, ones16, n_acc, jpt)

    x, m1 = _tc1(node_features, ffns["pre"], ffns["conv1_prep"],
                 inv_e, n, br)
    sums1 = _sc_scatter(m1, src2, dst2, zeros32, n_acc, jpt)
    xa, m2 = _tc_combine(x, sums1, cnts, ffns["conv1_upd"],
                         ffns["conv2_prep"], None, inv_e, n, br)
    sums2 = _sc_scatter(m2, src2, dst2, zeros32, n_acc, jpt)
    ncls = params["Wl"].shape[1]
    y_all = _tc_combine(xa, sums2, cnts, ffns["conv2_upd"], None,
                        (ffns["post"], params["Wl"],
                         params["bl"].reshape(1, -1)),
                        inv_e, n, br)[0]
    return _sc_gather(y_all, input_node_indices,
                      input_node_indices.shape[0], ncls)


# speedup vs baseline: 18.1709x; 1.1333x over previous
import functools

import jax
import jax.numpy as jnp
from jax import lax
from jax.experimental import pallas as pl
from jax.experimental.pallas import tpu as pltpu
from jax.experimental.pallas import tpu_sc as plsc

_NC = 2
_NS = 16
_NW = _NC * _NS

_SQRT2 = 1.4142135623730951


def _gelu(x):
    return x * 0.5 * (1.0 + lax.erf(x / _SQRT2))


def _bn_params(layer):
    s = layer["gamma"] / jnp.sqrt(layer["mvar"] + 1e-3)
    t = layer["beta"] - layer["mmean"] * s
    return [s.reshape(1, -1), t.reshape(1, -1), layer["W"],
            layer["b"].reshape(1, -1)]


def _layer(x, s, t, w, b):
    return _gelu(jnp.dot(x * s[...] + t[...], w[...],
                         preferred_element_type=jnp.float32) + b[...])


def _wspecs(arrs):
    return [pl.BlockSpec(a.shape, lambda i, r=len(a.shape): (0,) * r)
            for a in arrs]


_BR = 2048
_P = _BR // 4


def _pack4(v):
    return jnp.concatenate([v[0:_P], v[_P:2 * _P],
                            v[2 * _P:3 * _P], v[3 * _P:4 * _P]], axis=1)


def _unpack4(b):
    return jnp.concatenate([b[:, 0:32], b[:, 32:64],
                            b[:, 64:96], b[:, 96:128]], axis=0)


def _pack8w16(v):
    return jnp.concatenate([v[256 * k:256 * (k + 1)] for k in range(8)],
                           axis=1)


def _tc1(nf, pre, prep, inv_e, n, g):
    def body(nf_r, s1, t1, w1, b1, s2, t2, w2, b2,
             p1s, p1t, p1w, p1b, p2s, p2t, p2w, p2b, x_r, m_r):
        x = _layer(_layer(nf_r[...], s1, t1, w1, b1), s2, t2, w2, b2)
        x_r[...] = _pack4(x)
        p = _layer(_layer(x, p1s, p1t, p1w, p1b), p2s, p2t, p2w, p2b)
        m_r[...] = _pack4((p * inv_e).astype(jnp.bfloat16))

    d = nf.shape[1]
    ws = pre[0] + pre[1] + prep[0] + prep[1]
    return pl.pallas_call(
        body,
        grid=(g,),
        in_specs=[pl.BlockSpec((_BR, d), lambda i: (i, 0))] + _wspecs(ws),
        out_specs=[pl.BlockSpec((_P, 128), lambda i: (i, 0)),
                   pl.BlockSpec((_P, 128), lambda i: (i, 0))],
        out_shape=[jax.ShapeDtypeStruct((g * _P, 128), jnp.float32),
                   jax.ShapeDtypeStruct((g * _P, 128), jnp.bfloat16)],
        compiler_params=pltpu.CompilerParams(
            dimension_semantics=("parallel",)),
    )(nf, *ws)


def _tc_combine(x, sums, cnts, upd, prep, final, inv_e, g):
    with_prep = prep is not None

    def body(*refs):
        if with_prep:
            (x_r, s_r, c_r, u1s, u1t, u1w, u1b, u2s, u2t, u2w, u2b,
             p1s, p1t, p1w, p1b, p2s, p2t, p2w, p2b, xa_r, m_r) = refs
        else:
            (x_r, s_r, c_r, u1s, u1t, u1w, u1b, u2s, u2t, u2w, u2b,
             q1s, q1t, q1w, q1b, q2s, q2t, q2w, q2b, wlr, blr, y_r) = refs
        cc = c_r[0] + c_r[1]
        counts = jnp.maximum(
            jnp.concatenate([cc[:, 16 * k:16 * k + 1] for k in range(8)],
                            axis=0), 1.0)
        s = s_r[0].astype(jnp.float32) + s_r[1].astype(jnp.float32)
        agg = _unpack4(s) / counts
        x = _unpack4(x_r[...])
        h = jnp.concatenate([x, agg], axis=1)
        u = _layer(_layer(h, u1s, u1t, u1w, u1b), u2s, u2t, u2w, u2b)
        nrm = jnp.sqrt(jnp.maximum(jnp.sum(u * u, axis=1, keepdims=True),
                                   1e-12))
        xa = u / nrm + x
        if with_prep:
            xa_r[...] = _pack4(xa)
            p = _layer(_layer(xa, p1s, p1t, p1w, p1b), p2s, p2t, p2w, p2b)
            m_r[...] = _pack4((p * inv_e).astype(jnp.bfloat16))
        else:
            p = _layer(_layer(xa, q1s, q1t, q1w, q1b), q2s, q2t, q2w, q2b)
            y = jnp.dot(p, wlr[...],
                        preferred_element_type=jnp.float32) + blr[...]
            y_r[...] = _pack8w16(y)

    if with_prep:
        ws = upd[0] + upd[1] + prep[0] + prep[1]
    else:
        post, wl, bl = final
        ws = upd[0] + upd[1] + post[0] + post[1] + [wl, bl]
    in_specs = [
        pl.BlockSpec((_P, 128), lambda i: (i, 0)),
        pl.BlockSpec((2, _P, 128), lambda i: (0, i, 0)),
        pl.BlockSpec((2, 256, 128), lambda i: (0, i, 0)),
    ] + _wspecs(ws)
    if with_prep:
        out_specs = [pl.BlockSpec((_P, 128), lambda i: (i, 0))] * 2
        out_shape = [jax.ShapeDtypeStruct((g * _P, 128), jnp.float32),
                     jax.ShapeDtypeStruct((g * _P, 128), jnp.bfloat16)]
    else:
        out_specs = [pl.BlockSpec((256, 128), lambda i: (i, 0))]
        out_shape = [jax.ShapeDtypeStruct((g * 256, 128), jnp.float32)]

    res = pl.pallas_call(
        body,
        grid=(g,),
        in_specs=in_specs,
        out_specs=out_specs,
        out_shape=out_shape,
        compiler_params=pltpu.CompilerParams(
            dimension_semantics=("parallel",)),
    )(x, sums, cnts, *ws)
    return res


def _row_partition(n):
    rpt = ((n // _NS + 7) // 8) * 8
    last = n - (_NS - 1) * rpt
    assert last > 0 and last % 8 == 0 and rpt % 8 == 0
    return rpt, last


def _sc_counts(dst2, zeros16, ones_hbm, n_acc, jpt):
    rpt, last = _row_partition(n_acc)
    mesh = plsc.VectorSubcoreMesh(core_axis_name="c", subcore_axis_name="s")

    @functools.partial(
        pl.kernel,
        out_type=jax.ShapeDtypeStruct((_NC, n_acc, 16), jnp.float32),
        mesh=mesh,
        compiler_params=pltpu.CompilerParams(use_tc_tiling_on_sc=False),
        scratch_types=[
            pltpu.VMEM_SHARED((n_acc, 16), jnp.float32),
            pltpu.VMEM((jpt, 128), jnp.int32),
            pltpu.VMEM((128, 16), jnp.float32),
        ],
    )
    def kern(dst_hbm, z_hbm, o_hbm, out_hbm, acc, didx, ones_v):
        c = lax.axis_index("c")
        s = lax.axis_index("s")
        w = c * _NS + s
        pltpu.sync_copy(o_hbm, ones_v)
        pltpu.sync_copy(dst_hbm.at[pl.ds(w * jpt, jpt)], didx)

        @pl.when(s < _NS - 1)
        def _():
            pltpu.sync_copy(z_hbm, acc.at[pl.ds(s * rpt, rpt)])

        @pl.when(s == _NS - 1)
        def _():
            pltpu.sync_copy(z_hbm.at[pl.ds(0, last)],
                            acc.at[pl.ds(s * rpt, last)])

        plsc.subcore_barrier()

        def body(j, carry):
            pltpu.sync_copy(ones_v, acc.at[didx.at[j]], add=True)
            return carry

        lax.fori_loop(0, jpt, body, 0)
        plsc.subcore_barrier()

        @pl.when(s < _NS - 1)
        def _():
            pltpu.sync_copy(acc.at[pl.ds(s * rpt, rpt)],
                            out_hbm.at[c, pl.ds(s * rpt, rpt)])

        @pl.when(s == _NS - 1)
        def _():
            pltpu.sync_copy(acc.at[pl.ds(s * rpt, last)],
                            out_hbm.at[c, pl.ds(s * rpt, last)])

    return kern(dst2, zeros16, ones_hbm)


def _sc_scatter(m, src2, dst2, zeros32, n_acc, jpt):
    n = m.shape[0]
    rpt, last = _row_partition(n_acc)
    rptm, lastm = _row_partition(n)
    mesh = plsc.VectorSubcoreMesh(core_axis_name="c", subcore_axis_name="s")
    jb = next(cand for cand in range(min(32, jpt), 3, -1)
              if jpt % cand == 0 and cand % 4 == 0)

    @functools.partial(
        pl.kernel,
        out_type=jax.ShapeDtypeStruct((_NC, n_acc, 32), jnp.bfloat16),
        mesh=mesh,
        compiler_params=pltpu.CompilerParams(use_tc_tiling_on_sc=False),
        scratch_types=[
            pltpu.VMEM_SHARED((n_acc, 32), jnp.bfloat16),
            pltpu.VMEM_SHARED((n, 32), jnp.bfloat16),
            pltpu.VMEM((jb, 128), jnp.int32),
            pltpu.VMEM((jb, 128), jnp.int32),
            pltpu.VMEM((128, 32), jnp.bfloat16),
            pltpu.VMEM((128, 32), jnp.bfloat16),
            pltpu.VMEM((128, 32), jnp.bfloat16),
            pltpu.VMEM((128, 32), jnp.bfloat16),
            pltpu.SemaphoreType.DMA,
            pltpu.SemaphoreType.DMA,
            pltpu.SemaphoreType.DMA,
            pltpu.SemaphoreType.DMA,
            pltpu.SemaphoreType.DMA,
            pltpu.SemaphoreType.DMA,
            pltpu.SemaphoreType.DMA,
            pltpu.SemaphoreType.DMA,
        ],
    )
    def kern(m_hbm, src_hbm, dst_hbm, z_hbm, out_hbm,
             acc, mb, sidx, didx, r0, r1, r2, r3,
             g0, g1, g2, g3, t0, t1, t2, t3):
        bufs = (r0, r1, r2, r3)
        gsems = (g0, g1, g2, g3)
        ssems = (t0, t1, t2, t3)
        c = lax.axis_index("c")
        s = lax.axis_index("s")
        w = c * _NS + s

        @pl.when(s < _NS - 1)
        def _():
            pltpu.sync_copy(z_hbm, acc.at[pl.ds(s * rpt, rpt)])
            pltpu.sync_copy(m_hbm.at[pl.ds(s * rptm, rptm)],
                            mb.at[pl.ds(s * rptm, rptm)])

        @pl.when(s == _NS - 1)
        def _():
            pltpu.sync_copy(z_hbm.at[pl.ds(0, last)],
                            acc.at[pl.ds(s * rpt, last)])
            pltpu.sync_copy(m_hbm.at[pl.ds(s * rptm, lastm)],
                            mb.at[pl.ds(s * rptm, lastm)])

        plsc.subcore_barrier()

        def block(bi, bcarry):
            row0 = w * jpt + bi * jb
            pltpu.sync_copy(src_hbm.at[pl.ds(row0, jb)], sidx)
            pltpu.sync_copy(dst_hbm.at[pl.ds(row0, jb)], didx)
            for l in range(3):
                pltpu.async_copy(mb.at[sidx.at[l]], bufs[l], gsems[l])

            def quad(t, carry):
                for l in range(4):
                    j = 4 * t + l
                    pltpu.make_async_copy(mb.at[sidx.at[j]],
                                          bufs[l], gsems[l]).wait()
                    pltpu.async_copy(bufs[l], acc.at[didx.at[j]],
                                     ssems[l], add=True)

                    @pl.when(j + 3 < jb)
                    def _():
                        ln = (l + 3) % 4

                        @pl.when(j > 0)
                        def _():
                            pltpu.make_async_copy(
                                bufs[ln], acc.at[didx.at[j]],
                                ssems[ln]).wait()

                        pltpu.async_copy(mb.at[sidx.at[j + 3]],
                                         bufs[ln], gsems[ln])
                return carry

            lax.fori_loop(0, jb // 4, quad, 0)
            for l in range(4):
                pltpu.make_async_copy(bufs[l], acc.at[didx.at[l]],
                                      ssems[l]).wait()
            return bcarry

        lax.fori_loop(0, jpt // jb, block, 0)
        plsc.subcore_barrier()

        @pl.when(s < _NS - 1)
        def _():
            pltpu.sync_copy(acc.at[pl.ds(s * rpt, rpt)],
                            out_hbm.at[c, pl.ds(s * rpt, rpt)])

        @pl.when(s == _NS - 1)
        def _():
            pltpu.sync_copy(acc.at[pl.ds(s * rpt, last)],
                            out_hbm.at[c, pl.ds(s * rpt, last)])

    return kern(m, src2, dst2, zeros32)


def _sc_gather(xb, idx, b, width):
    bpw = b // _NW
    mesh = plsc.VectorSubcoreMesh(core_axis_name="c", subcore_axis_name="s")

    @functools.partial(
        pl.kernel,
        out_type=jax.ShapeDtypeStruct((b, width), jnp.float32),
        mesh=mesh,
        compiler_params=pltpu.CompilerParams(use_tc_tiling_on_sc=False),
        scratch_types=[
            pltpu.VMEM((bpw,), jnp.int32),
            pltpu.VMEM((bpw, width), jnp.float32),
            pltpu.SemaphoreType.DMA,
        ],
    )
    def kern(x_hbm, idx_hbm, out_hbm, idxv, rows, sem):
        c = lax.axis_index("c")
        s = lax.axis_index("s")
        base = (s * _NC + c) * bpw
        pltpu.sync_copy(idx_hbm.at[pl.ds(base, bpw)], idxv)
        pltpu.async_copy(x_hbm.at[idxv], rows, sem).wait()
        pltpu.sync_copy(rows, out_hbm.at[pl.ds(base, bpw)])

    return kern(xb, idx)


def _row32(v):
    u = v % _BR
    return _BR * (v // _BR) + 4 * (u % _P) + u // _P


def _row16(v):
    u = v % _BR
    return 8 * (256 * (v // _BR) + u % 256) + u // 256


def kernel(node_features, params, edges, input_node_indices):
    n, d = node_features.shape
    e = edges.shape[1]
    inv_e = 1.0 / float(e)
    g = -(-n // _BR)
    n2 = g * _BR
    n_acc = n2 + 64

    ffns = {name: [_bn_params(l) for l in params[name]]
            for name in ("pre", "conv1_prep", "conv1_upd",
                         "conv2_prep", "conv2_upd", "post")}

    jpt = -(-e // (_NW * 128))
    jpt += jpt % 2
    e_tot = _NW * 128 * jpt
    dstp = jnp.concatenate(
        [edges[0], jnp.full((e_tot - e,), n2, jnp.int32)])
    srcp = jnp.concatenate(
        [edges[1], jnp.zeros((e_tot - e,), jnp.int32)])
    src2 = _row32(srcp).reshape(-1, 128)
    dst2 = _row32(dstp).reshape(-1, 128)
    dstc2 = _row16(dstp).reshape(-1, 128)
    rpt, _ = _row_partition(n_acc)
    zeros32 = jnp.zeros((rpt, 32), jnp.bfloat16)
    zeros16 = jnp.zeros((rpt, 16), jnp.float32)
    ones16 = jnp.ones((128, 16), jnp.float32)

    cnts = _sc_counts(dstc2, zeros16, ones16, n_acc, jpt)
    cnts_p = cnts.reshape(_NC, n_acc // 8, 128)

    x_p, m1_p = _tc1(node_features, ffns["pre"], ffns["conv1_prep"],
                     inv_e, n, g)
    sums1 = _sc_scatter(m1_p.reshape(n2, 32), src2, dst2, zeros32,
                        n_acc, jpt)
    xa_p, m2_p = _tc_combine(x_p, sums1.reshape(_NC, n_acc // 4, 128),
                             cnts_p, ffns["conv1_upd"],
                             ffns["conv2_prep"], None, inv_e, g)
    sums2 = _sc_scatter(m2_p.reshape(n2, 32), src2, dst2, zeros32,
                        n_acc, jpt)
    ncls = params["Wl"].shape[1]
    y_p = _tc_combine(xa_p, sums2.reshape(_NC, n_acc // 4, 128),
                      cnts_p, ffns["conv2_upd"], None,
                      (ffns["post"], params["Wl"],
                       params["bl"].reshape(1, -1)),
                      inv_e, g)[0]
    return _sc_gather(y_p.reshape(n2, 16), _row16(input_node_indices),
                      input_node_indices.shape[0], ncls)


# speedup vs baseline: 19.8346x; 1.0916x over previous
import functools

import jax
import jax.numpy as jnp
from jax import lax
from jax.experimental import pallas as pl
from jax.experimental.pallas import tpu as pltpu
from jax.experimental.pallas import tpu_sc as plsc

_NC = 2
_NS = 16
_NW = _NC * _NS

_SQRT2 = 1.4142135623730951


def _gelu(x):
    return x * 0.5 * (1.0 + lax.erf(x / _SQRT2))


def _bn_params(layer):
    s = layer["gamma"] / jnp.sqrt(layer["mvar"] + 1e-3)
    t = layer["beta"] - layer["mmean"] * s
    return [s.reshape(1, -1), t.reshape(1, -1), layer["W"],
            layer["b"].reshape(1, -1)]


def _layer(x, s, t, w, b):
    return _gelu(jnp.dot(x * s[...] + t[...], w[...],
                         preferred_element_type=jnp.float32) + b[...])


def _wspecs(arrs):
    return [pl.BlockSpec(a.shape, lambda i, r=len(a.shape): (0,) * r)
            for a in arrs]


_BR = 2048
_P = _BR // 4


def _pack4(v):
    return jnp.concatenate([v[0:_P], v[_P:2 * _P],
                            v[2 * _P:3 * _P], v[3 * _P:4 * _P]], axis=1)


def _bd(w):
    return jax.scipy.linalg.block_diag(w, w, w, w)


def _bd_layer(lp):
    s, t, w, b = lp
    return [jnp.tile(s, (1, 4)), jnp.tile(t, (1, 4)), _bd(w),
            jnp.tile(b, (1, 4))]


def _bd_upd(lp):
    s, t, w, b = lp
    return [jnp.tile(s[:, :32], (1, 4)), jnp.tile(t[:, :32], (1, 4)),
            _bd(w[:32]),
            jnp.tile(s[:, 32:], (1, 4)), jnp.tile(t[:, 32:], (1, 4)),
            _bd(w[32:]),
            jnp.tile(b, (1, 4))]


def _tc1(nf, pre, prep, inv_e, n, g):
    def body(nf_r, s1, t1, w1, b1, s2, t2, w2, b2,
             p1s, p1t, p1w, p1b, p2s, p2t, p2w, p2b, x_r, m_r):
        h1 = _layer(nf_r[...], s1, t1, w1, b1)
        x_p = _layer(_pack4(h1), s2, t2, w2, b2)
        x_r[...] = x_p
        p = _layer(_layer(x_p, p1s, p1t, p1w, p1b), p2s, p2t, p2w, p2b)
        m_r[...] = (p * inv_e).astype(jnp.bfloat16)

    d = nf.shape[1]
    ws = pre[0] + _bd_layer(pre[1]) + _bd_layer(prep[0]) + _bd_layer(prep[1])
    return pl.pallas_call(
        body,
        grid=(g,),
        in_specs=[pl.BlockSpec((_BR, d), lambda i: (i, 0))] + _wspecs(ws),
        out_specs=[pl.BlockSpec((_P, 128), lambda i: (i, 0)),
                   pl.BlockSpec((_P, 128), lambda i: (i, 0))],
        out_shape=[jax.ShapeDtypeStruct((g * _P, 128), jnp.float32),
                   jax.ShapeDtypeStruct((g * _P, 128), jnp.bfloat16)],
        compiler_params=pltpu.CompilerParams(
            dimension_semantics=("parallel",)),
    )(nf, *ws)


def _tc_combine(x, sums, cnts, upd, prep, final, inv_e, g):
    with_prep = prep is not None
    ones_bd = _bd(jnp.ones((32, 32), jnp.float32))

    def body(*refs):
        if with_prep:
            (x_r, s_r, c_r, ob_r,
             usx, utx, uwa, usa, uta, uwb, ub1, u2s, u2t, u2w, u2b,
             p1s, p1t, p1w, p1b, p2s, p2t, p2w, p2b,
             xa_r, m_r, cm_r) = refs
            cc = jnp.maximum(c_r[0] + c_r[1], 1.0)
            cm_r[...] = cc
        else:
            (x_r, s_r, c_r, ob_r,
             usx, utx, uwa, usa, uta, uwb, ub1, u2s, u2t, u2w, u2b,
             q1s, q1t, q1w, q1b, q2s, q2t, q2w, q2b, wlr, blr,
             y_r) = refs
            cc = c_r[...]
        s = s_r[0].astype(jnp.float32) + s_r[1].astype(jnp.float32)
        agg = s / cc
        x = x_r[...]
        u = _gelu(jnp.dot(x * usx[...] + utx[...], uwa[...],
                          preferred_element_type=jnp.float32)
                  + jnp.dot(agg * usa[...] + uta[...], uwb[...],
                            preferred_element_type=jnp.float32) + ub1[...])
        u = _layer(u, u2s, u2t, u2w, u2b)
        nrm2 = jnp.dot(u * u, ob_r[...],
                       preferred_element_type=jnp.float32,
                       precision=lax.Precision.HIGHEST)
        xa = u / jnp.sqrt(jnp.maximum(nrm2, 1e-12)) + x
        if with_prep:
            xa_r[...] = xa
            p = _layer(_layer(xa, p1s, p1t, p1w, p1b), p2s, p2t, p2w, p2b)
            m_r[...] = (p * inv_e).astype(jnp.bfloat16)
        else:
            p = _layer(_layer(xa, q1s, q1t, q1w, q1b), q2s, q2t, q2w, q2b)
            y_r[...] = jnp.dot(p, wlr[...],
                               preferred_element_type=jnp.float32) + blr[...]

    if with_prep:
        ws = _bd_upd(upd[0]) + _bd_layer(upd[1]) \
            + _bd_layer(prep[0]) + _bd_layer(prep[1])
        cspec = pl.BlockSpec((2, _P, 128), lambda i: (0, i, 0))
    else:
        post, wl, bl = final
        wlp = jnp.zeros((32, 32), jnp.float32).at[:, :wl.shape[1]].set(wl)
        blp = jnp.zeros((1, 32), jnp.float32).at[:, :wl.shape[1]].set(bl)
        ws = _bd_upd(upd[0]) + _bd_layer(upd[1]) \
            + _bd_layer(post[0]) + _bd_layer(post[1]) \
            + [_bd(wlp), jnp.tile(blp, (1, 4))]
        cspec = pl.BlockSpec((_P, 128), lambda i: (i, 0))
    in_specs = [
        pl.BlockSpec((_P, 128), lambda i: (i, 0)),
        pl.BlockSpec((2, _P, 128), lambda i: (0, i, 0)),
        cspec,
        pl.BlockSpec((128, 128), lambda i: (0, 0)),
    ] + _wspecs(ws)
    if with_prep:
        out_specs = [pl.BlockSpec((_P, 128), lambda i: (i, 0))] * 3
        out_shape = [jax.ShapeDtypeStruct((g * _P, 128), jnp.float32),
                     jax.ShapeDtypeStruct((g * _P, 128), jnp.bfloat16),
                     jax.ShapeDtypeStruct((g * _P, 128), jnp.float32)]
    else:
        out_specs = [pl.BlockSpec((_P, 128), lambda i: (i, 0))]
        out_shape = [jax.ShapeDtypeStruct((g * _P, 128), jnp.float32)]

    res = pl.pallas_call(
        body,
        grid=(g,),
        in_specs=in_specs,
        out_specs=out_specs,
        out_shape=out_shape,
        compiler_params=pltpu.CompilerParams(
            dimension_semantics=("parallel",)),
    )(x, sums, cnts, ones_bd, *ws)
    return res


def _row_partition(n):
    rpt = ((n // _NS + 7) // 8) * 8
    last = n - (_NS - 1) * rpt
    assert last > 0 and last % 8 == 0 and rpt % 8 == 0
    return rpt, last


def _sc_counts(dst2, zeros16, ones_hbm, n_acc, jpt):
    rpt, last = _row_partition(n_acc)
    mesh = plsc.VectorSubcoreMesh(core_axis_name="c", subcore_axis_name="s")

    jb = next(cand for cand in range(min(32, jpt), 3, -1)
              if jpt % cand == 0 and cand % 4 == 0)

    @functools.partial(
        pl.kernel,
        out_type=jax.ShapeDtypeStruct((_NC, n_acc, 32), jnp.float32),
        mesh=mesh,
        compiler_params=pltpu.CompilerParams(use_tc_tiling_on_sc=False),
        scratch_types=[
            pltpu.VMEM_SHARED((n_acc, 32), jnp.float32),
            pltpu.VMEM((jb, 128), jnp.int32),
            pltpu.VMEM((128, 32), jnp.float32),
        ],
    )
    def kern(dst_hbm, z_hbm, o_hbm, out_hbm, acc, didx, ones_v):
        c = lax.axis_index("c")
        s = lax.axis_index("s")
        w = c * _NS + s
        pltpu.sync_copy(o_hbm, ones_v)

        @pl.when(s < _NS - 1)
        def _():
            pltpu.sync_copy(z_hbm, acc.at[pl.ds(s * rpt, rpt)])

        @pl.when(s == _NS - 1)
        def _():
            pltpu.sync_copy(z_hbm.at[pl.ds(0, last)],
                            acc.at[pl.ds(s * rpt, last)])

        plsc.subcore_barrier()

        def block(bi, bcarry):
            pltpu.sync_copy(dst_hbm.at[pl.ds(w * jpt + bi * jb, jb)], didx)

            def body(j, carry):
                pltpu.sync_copy(ones_v, acc.at[didx.at[j]], add=True)
                return carry

            lax.fori_loop(0, jb, body, 0)
            return bcarry

        lax.fori_loop(0, jpt // jb, block, 0)
        plsc.subcore_barrier()

        @pl.when(s < _NS - 1)
        def _():
            pltpu.sync_copy(acc.at[pl.ds(s * rpt, rpt)],
                            out_hbm.at[c, pl.ds(s * rpt, rpt)])

        @pl.when(s == _NS - 1)
        def _():
            pltpu.sync_copy(acc.at[pl.ds(s * rpt, last)],
                            out_hbm.at[c, pl.ds(s * rpt, last)])

    return kern(dst2, zeros16, ones_hbm)


def _sc_scatter(m, src2, dst2, zeros32, n_acc, jpt):
    n = m.shape[0]
    rpt, last = _row_partition(n_acc)
    rptm, lastm = _row_partition(n)
    mesh = plsc.VectorSubcoreMesh(core_axis_name="c", subcore_axis_name="s")
    jb = next(cand for cand in range(min(32, jpt), 3, -1)
              if jpt % cand == 0 and cand % 4 == 0)

    @functools.partial(
        pl.kernel,
        out_type=jax.ShapeDtypeStruct((_NC, n_acc, 32), jnp.bfloat16),
        mesh=mesh,
        compiler_params=pltpu.CompilerParams(use_tc_tiling_on_sc=False),
        scratch_types=[
            pltpu.VMEM_SHARED((n_acc, 32), jnp.bfloat16),
            pltpu.VMEM_SHARED((n, 32), jnp.bfloat16),
            pltpu.VMEM((jb, 128), jnp.int32),
            pltpu.VMEM((jb, 128), jnp.int32),
            pltpu.VMEM((128, 32), jnp.bfloat16),
            pltpu.VMEM((128, 32), jnp.bfloat16),
            pltpu.VMEM((128, 32), jnp.bfloat16),
            pltpu.VMEM((128, 32), jnp.bfloat16),
            pltpu.SemaphoreType.DMA,
            pltpu.SemaphoreType.DMA,
            pltpu.SemaphoreType.DMA,
            pltpu.SemaphoreType.DMA,
            pltpu.SemaphoreType.DMA,
            pltpu.SemaphoreType.DMA,
            pltpu.SemaphoreType.DMA,
            pltpu.SemaphoreType.DMA,
        ],
    )
    def kern(m_hbm, src_hbm, dst_hbm, z_hbm, out_hbm,
             acc, mb, sidx, didx, r0, r1, r2, r3,
             g0, g1, g2, g3, t0, t1, t2, t3):
        bufs = (r0, r1, r2, r3)
        gsems = (g0, g1, g2, g3)
        ssems = (t0, t1, t2, t3)
        c = lax.axis_index("c")
        s = lax.axis_index("s")
        w = c * _NS + s

        @pl.when(s < _NS - 1)
        def _():
            pltpu.sync_copy(z_hbm, acc.at[pl.ds(s * rpt, rpt)])
            pltpu.sync_copy(m_hbm.at[pl.ds(s * rptm, rptm)],
                            mb.at[pl.ds(s * rptm, rptm)])

        @pl.when(s == _NS - 1)
        def _():
            pltpu.sync_copy(z_hbm.at[pl.ds(0, last)],
                            acc.at[pl.ds(s * rpt, last)])
            pltpu.sync_copy(m_hbm.at[pl.ds(s * rptm, lastm)],
                            mb.at[pl.ds(s * rptm, lastm)])

        plsc.subcore_barrier()

        def block(bi, bcarry):
            row0 = w * jpt + bi * jb
            pltpu.sync_copy(src_hbm.at[pl.ds(row0, jb)], sidx)
            pltpu.sync_copy(dst_hbm.at[pl.ds(row0, jb)], didx)
            for l in range(3):
                pltpu.async_copy(mb.at[sidx.at[l]], bufs[l], gsems[l])

            def quad(t, carry):
                for l in range(4):
                    j = 4 * t + l
                    pltpu.make_async_copy(mb.at[sidx.at[j]],
                                          bufs[l], gsems[l]).wait()
                    pltpu.async_copy(bufs[l], acc.at[didx.at[j]],
                                     ssems[l], add=True)

                    @pl.when(j + 3 < jb)
                    def _():
                        ln = (l + 3) % 4

                        @pl.when(j > 0)
                        def _():
                            pltpu.make_async_copy(
                                bufs[ln], acc.at[didx.at[j]],
                                ssems[ln]).wait()

                        pltpu.async_copy(mb.at[sidx.at[j + 3]],
                                         bufs[ln], gsems[ln])
                return carry

            lax.fori_loop(0, jb // 4, quad, 0)
            for l in range(4):
                pltpu.make_async_copy(bufs[l], acc.at[didx.at[l]],
                                      ssems[l]).wait()
            return bcarry

        lax.fori_loop(0, jpt // jb, block, 0)
        plsc.subcore_barrier()

        @pl.when(s < _NS - 1)
        def _():
            pltpu.sync_copy(acc.at[pl.ds(s * rpt, rpt)],
                            out_hbm.at[c, pl.ds(s * rpt, rpt)])

        @pl.when(s == _NS - 1)
        def _():
            pltpu.sync_copy(acc.at[pl.ds(s * rpt, last)],
                            out_hbm.at[c, pl.ds(s * rpt, last)])

    return kern(m, src2, dst2, zeros32)


def _sc_gather(xb, idx, b, width):
    bpw = b // _NW
    mesh = plsc.VectorSubcoreMesh(core_axis_name="c", subcore_axis_name="s")

    @functools.partial(
        pl.kernel,
        out_type=jax.ShapeDtypeStruct((b, width), jnp.float32),
        mesh=mesh,
        compiler_params=pltpu.CompilerParams(use_tc_tiling_on_sc=False),
        scratch_types=[
            pltpu.VMEM((bpw,), jnp.int32),
            pltpu.VMEM((bpw, width), jnp.float32),
            pltpu.SemaphoreType.DMA,
        ],
    )
    def kern(x_hbm, idx_hbm, out_hbm, idxv, rows, sem):
        c = lax.axis_index("c")
        s = lax.axis_index("s")
        base = (s * _NC + c) * bpw
        pltpu.sync_copy(idx_hbm.at[pl.ds(base, bpw)], idxv)
        pltpu.async_copy(x_hbm.at[idxv], rows, sem).wait()
        pltpu.sync_copy(rows, out_hbm.at[pl.ds(base, bpw)])

    return kern(xb, idx)


def _row32(v):
    u = v % _BR
    return _BR * (v // _BR) + 4 * (u % _P) + u // _P


def _row16(v):
    u = v % _BR
    return 8 * (256 * (v // _BR) + u % 256) + u // 256


def kernel(node_features, params, edges, input_node_indices):
    n, d = node_features.shape
    e = edges.shape[1]
    inv_e = 1.0 / float(e)
    g = -(-n // _BR)
    n2 = g * _BR
    n_acc = n2 + 64

    ffns = {name: [_bn_params(l) for l in params[name]]
            for name in ("pre", "conv1_prep", "conv1_upd",
                         "conv2_prep", "conv2_upd", "post")}

    jpt = -(-e // (_NW * 128))
    jpt += jpt % 2
    e_tot = _NW * 128 * jpt
    dstp = jnp.concatenate(
        [edges[0], jnp.full((e_tot - e,), n2, jnp.int32)])
    srcp = jnp.concatenate(
        [edges[1], jnp.zeros((e_tot - e,), jnp.int32)])
    src2 = _row32(srcp).reshape(-1, 128)
    dst2 = _row32(dstp).reshape(-1, 128)
    rpt, _ = _row_partition(n_acc)
    zeros32b = jnp.zeros((rpt, 32), jnp.bfloat16)
    zeros32f = jnp.zeros((rpt, 32), jnp.float32)
    ones32 = jnp.ones((128, 32), jnp.float32)

    cnts = _sc_counts(dst2, zeros32f, ones32, n_acc, jpt)
    cnts_p = cnts.reshape(_NC, n_acc // 4, 128)

    x_p, m1_p = _tc1(node_features, ffns["pre"], ffns["conv1_prep"],
                     inv_e, n, g)
    sums1 = _sc_scatter(m1_p.reshape(n2, 32), src2, dst2, zeros32b,
                        n_acc, jpt)
    xa_p, m2_p, ccm = _tc_combine(x_p, sums1.reshape(_NC, n_acc // 4, 128),
                                  cnts_p, ffns["conv1_upd"],
                                  ffns["conv2_prep"], None, inv_e, g)
    sums2 = _sc_scatter(m2_p.reshape(n2, 32), src2, dst2, zeros32b,
                        n_acc, jpt)
    ncls = params["Wl"].shape[1]
    y_p = _tc_combine(xa_p, sums2.reshape(_NC, n_acc // 4, 128),
                      ccm, ffns["conv2_upd"], None,
                      (ffns["post"], params["Wl"],
                       params["bl"].reshape(1, -1)),
                      inv_e, g)[0]
    out = _sc_gather(y_p.reshape(n2, 32), _row32(input_node_indices),
                     input_node_indices.shape[0], 32)
    return out[:, :ncls]


# speedup vs baseline: 20.4597x; 1.0315x over previous
import functools

import jax
import jax.numpy as jnp
from jax import lax
from jax.experimental import pallas as pl
from jax.experimental.pallas import tpu as pltpu
from jax.experimental.pallas import tpu_sc as plsc

_NC = 2
_NS = 16
_NW = _NC * _NS

_SQRT2 = 1.4142135623730951


def _gelu(x):
    return x * 0.5 * (1.0 + lax.erf(x / _SQRT2))


def _bn_params(layer):
    s = layer["gamma"] / jnp.sqrt(layer["mvar"] + 1e-3)
    t = layer["beta"] - layer["mmean"] * s
    return [s.reshape(1, -1), t.reshape(1, -1), layer["W"],
            layer["b"].reshape(1, -1)]


def _layer(x, s, t, w, b):
    return _gelu(jnp.dot(x * s[...] + t[...], w[...],
                         preferred_element_type=jnp.float32) + b[...])


def _wspecs(arrs):
    return [pl.BlockSpec(a.shape, lambda i, r=len(a.shape): (0,) * r)
            for a in arrs]


_BR = 2048
_P = _BR // 4


def _pack4(v):
    return jnp.concatenate([v[0:_P], v[_P:2 * _P],
                            v[2 * _P:3 * _P], v[3 * _P:4 * _P]], axis=1)


def _bd(w):
    return jax.scipy.linalg.block_diag(w, w, w, w)


def _bd_layer(lp):
    s, t, w, b = lp
    return [jnp.tile(s, (1, 4)), jnp.tile(t, (1, 4)), _bd(w),
            jnp.tile(b, (1, 4))]


def _bd_upd(lp):
    s, t, w, b = lp
    return [jnp.tile(s[:, :32], (1, 4)), jnp.tile(t[:, :32], (1, 4)),
            _bd(w[:32]),
            jnp.tile(s[:, 32:], (1, 4)), jnp.tile(t[:, 32:], (1, 4)),
            _bd(w[32:]),
            jnp.tile(b, (1, 4))]


def _tc1(nf, pre, prep, inv_e, n, g):
    def body(nf_r, s1, t1, w1, b1, s2, t2, w2, b2,
             p1s, p1t, p1w, p1b, p2s, p2t, p2w, p2b, x_r, m_r):
        h1 = _layer(nf_r[...], s1, t1, w1, b1)
        x_p = _layer(_pack4(h1), s2, t2, w2, b2)
        x_r[...] = x_p
        p = _layer(_layer(x_p, p1s, p1t, p1w, p1b), p2s, p2t, p2w, p2b)
        m_r[...] = (p * inv_e).astype(jnp.bfloat16)

    d = nf.shape[1]
    ws = pre[0] + _bd_layer(pre[1]) + _bd_layer(prep[0]) + _bd_layer(prep[1])
    return pl.pallas_call(
        body,
        grid=(g,),
        in_specs=[pl.BlockSpec((_BR, d), lambda i: (i, 0))] + _wspecs(ws),
        out_specs=[pl.BlockSpec((_P, 128), lambda i: (i, 0)),
                   pl.BlockSpec((_P, 128), lambda i: (i, 0))],
        out_shape=[jax.ShapeDtypeStruct((g * _P, 128), jnp.float32),
                   jax.ShapeDtypeStruct((g * _P, 128), jnp.bfloat16)],
        compiler_params=pltpu.CompilerParams(
            dimension_semantics=("parallel",)),
    )(nf, *ws)


def _tc_combine(x, sums, cnts, upd, prep, final, inv_e, g):
    with_prep = prep is not None
    ones_bd = _bd(jnp.ones((32, 32), jnp.float32))

    def body(*refs):
        if with_prep:
            (x_r, s_r, c_r, ob_r,
             usx, utx, uwa, usa, uta, uwb, ub1, u2s, u2t, u2w, u2b,
             p1s, p1t, p1w, p1b, p2s, p2t, p2w, p2b,
             xa_r, m_r, cm_r) = refs
            cc = jnp.maximum(c_r[0] + c_r[1], 1.0)
            cm_r[...] = cc
        else:
            (x_r, s_r, c_r, ob_r,
             usx, utx, uwa, usa, uta, uwb, ub1, u2s, u2t, u2w, u2b,
             q1s, q1t, q1w, q1b, q2s, q2t, q2w, q2b, wlr, blr,
             y_r) = refs
            cc = c_r[...]
        s = s_r[0].astype(jnp.float32) + s_r[1].astype(jnp.float32)
        agg = s / cc
        x = x_r[...]
        u = _gelu(jnp.dot(x * usx[...] + utx[...], uwa[...],
                          preferred_element_type=jnp.float32)
                  + jnp.dot(agg * usa[...] + uta[...], uwb[...],
                            preferred_element_type=jnp.float32) + ub1[...])
        u = _layer(u, u2s, u2t, u2w, u2b)
        nrm2 = jnp.dot(u * u, ob_r[...],
                       preferred_element_type=jnp.float32,
                       precision=lax.Precision.HIGHEST)
        xa = u / jnp.sqrt(jnp.maximum(nrm2, 1e-12)) + x
        if with_prep:
            xa_r[...] = xa
            p = _layer(_layer(xa, p1s, p1t, p1w, p1b), p2s, p2t, p2w, p2b)
            m_r[...] = (p * inv_e).astype(jnp.bfloat16)
        else:
            p = _layer(_layer(xa, q1s, q1t, q1w, q1b), q2s, q2t, q2w, q2b)
            y_r[...] = jnp.dot(p, wlr[...],
                               preferred_element_type=jnp.float32) + blr[...]

    if with_prep:
        ws = _bd_upd(upd[0]) + _bd_layer(upd[1]) \
            + _bd_layer(prep[0]) + _bd_layer(prep[1])
        cspec = pl.BlockSpec((2, _P, 128), lambda i: (0, i, 0))
    else:
        post, wl, bl = final
        wlp = jnp.zeros((32, 32), jnp.float32).at[:, :wl.shape[1]].set(wl)
        blp = jnp.zeros((1, 32), jnp.float32).at[:, :wl.shape[1]].set(bl)
        ws = _bd_upd(upd[0]) + _bd_layer(upd[1]) \
            + _bd_layer(post[0]) + _bd_layer(post[1]) \
            + [_bd(wlp), jnp.tile(blp, (1, 4))]
        cspec = pl.BlockSpec((_P, 128), lambda i: (i, 0))
    in_specs = [
        pl.BlockSpec((_P, 128), lambda i: (i, 0)),
        pl.BlockSpec((2, _P, 128), lambda i: (0, i, 0)),
        cspec,
        pl.BlockSpec((128, 128), lambda i: (0, 0)),
    ] + _wspecs(ws)
    if with_prep:
        out_specs = [pl.BlockSpec((_P, 128), lambda i: (i, 0))] * 3
        out_shape = [jax.ShapeDtypeStruct((g * _P, 128), jnp.float32),
                     jax.ShapeDtypeStruct((g * _P, 128), jnp.bfloat16),
                     jax.ShapeDtypeStruct((g * _P, 128), jnp.float32)]
    else:
        out_specs = [pl.BlockSpec((_P, 128), lambda i: (i, 0))]
        out_shape = [jax.ShapeDtypeStruct((g * _P, 128), jnp.float32)]

    res = pl.pallas_call(
        body,
        grid=(g,),
        in_specs=in_specs,
        out_specs=out_specs,
        out_shape=out_shape,
        compiler_params=pltpu.CompilerParams(
            dimension_semantics=("parallel",)),
    )(x, sums, cnts, ones_bd, *ws)
    return res


def _row_partition(n):
    rpt = ((n // _NS + 7) // 8) * 8
    last = n - (_NS - 1) * rpt
    assert last > 0 and last % 8 == 0 and rpt % 8 == 0
    return rpt, last


def _sc_counts(dst2, zeros16, ones_hbm, n_acc, jpt):
    rpt, last = _row_partition(n_acc)
    mesh = plsc.VectorSubcoreMesh(core_axis_name="c", subcore_axis_name="s")

    jb = next(cand for cand in range(min(32, jpt), 3, -1)
              if jpt % cand == 0 and cand % 4 == 0)

    @functools.partial(
        pl.kernel,
        out_type=jax.ShapeDtypeStruct((_NC, n_acc, 32), jnp.float32),
        mesh=mesh,
        compiler_params=pltpu.CompilerParams(use_tc_tiling_on_sc=False),
        scratch_types=[
            pltpu.VMEM_SHARED((n_acc, 32), jnp.float32),
            pltpu.VMEM((jb, 128), jnp.int32),
            pltpu.VMEM((128, 32), jnp.float32),
        ],
    )
    def kern(dst_hbm, z_hbm, o_hbm, out_hbm, acc, didx, ones_v):
        c = lax.axis_index("c")
        s = lax.axis_index("s")
        w = c * _NS + s
        pltpu.sync_copy(o_hbm, ones_v)

        @pl.when(s < _NS - 1)
        def _():
            pltpu.sync_copy(z_hbm, acc.at[pl.ds(s * rpt, rpt)])

        @pl.when(s == _NS - 1)
        def _():
            pltpu.sync_copy(z_hbm.at[pl.ds(0, last)],
                            acc.at[pl.ds(s * rpt, last)])

        plsc.subcore_barrier()

        def block(bi, bcarry):
            pltpu.sync_copy(dst_hbm.at[pl.ds(w * jpt + bi * jb, jb)], didx)

            def body(j, carry):
                pltpu.sync_copy(ones_v, acc.at[didx.at[j]], add=True)
                return carry

            lax.fori_loop(0, jb, body, 0)
            return bcarry

        lax.fori_loop(0, jpt // jb, block, 0)
        plsc.subcore_barrier()

        @pl.when(s < _NS - 1)
        def _():
            pltpu.sync_copy(acc.at[pl.ds(s * rpt, rpt)],
                            out_hbm.at[c, pl.ds(s * rpt, rpt)])

        @pl.when(s == _NS - 1)
        def _():
            pltpu.sync_copy(acc.at[pl.ds(s * rpt, last)],
                            out_hbm.at[c, pl.ds(s * rpt, last)])

    return kern(dst2, zeros16, ones_hbm)


def _sc_scatter(m, src2, dst2, zeros32, n_acc, jpt):
    n = m.shape[0]
    rpt, last = _row_partition(n_acc)
    rptm, lastm = _row_partition(n)
    mesh = plsc.VectorSubcoreMesh(core_axis_name="c", subcore_axis_name="s")
    jb = next(cand for cand in range(min(32, jpt), 3, -1)
              if jpt % cand == 0 and cand % 4 == 0)
    nb = jpt // jb

    @functools.partial(
        pl.kernel,
        out_type=jax.ShapeDtypeStruct((_NC, n_acc, 32), jnp.bfloat16),
        mesh=mesh,
        compiler_params=pltpu.CompilerParams(use_tc_tiling_on_sc=False),
        scratch_types=[
            pltpu.VMEM_SHARED((n_acc, 32), jnp.bfloat16),
            pltpu.VMEM_SHARED((n, 32), jnp.bfloat16),
            pltpu.VMEM((jb, 128), jnp.int32),
            pltpu.VMEM((jb, 128), jnp.int32),
            pltpu.VMEM((jb, 128), jnp.int32),
            pltpu.VMEM((jb, 128), jnp.int32),
            pltpu.VMEM((128, 32), jnp.bfloat16),
            pltpu.VMEM((128, 32), jnp.bfloat16),
            pltpu.VMEM((128, 32), jnp.bfloat16),
            pltpu.VMEM((128, 32), jnp.bfloat16),
            pltpu.SemaphoreType.DMA,
            pltpu.SemaphoreType.DMA,
            pltpu.SemaphoreType.DMA,
            pltpu.SemaphoreType.DMA,
            pltpu.SemaphoreType.DMA,
            pltpu.SemaphoreType.DMA,
            pltpu.SemaphoreType.DMA,
            pltpu.SemaphoreType.DMA,
            pltpu.SemaphoreType.DMA,
            pltpu.SemaphoreType.DMA,
            pltpu.SemaphoreType.DMA,
        ],
    )
    def kern(m_hbm, src_hbm, dst_hbm, z_hbm, out_hbm,
             acc, mb, si0, di0, si1, di1, r0, r1, r2, r3,
             g0, g1, g2, g3, t0, t1, t2, t3, zs, ms, isem):
        bufs = (r0, r1, r2, r3)
        gsems = (g0, g1, g2, g3)
        ssems = (t0, t1, t2, t3)
        c = lax.axis_index("c")
        s = lax.axis_index("s")
        w = c * _NS + s

        @pl.when(s < _NS - 1)
        def _():
            pltpu.async_copy(z_hbm, acc.at[pl.ds(s * rpt, rpt)], zs)
            pltpu.async_copy(m_hbm.at[pl.ds(s * rptm, rptm)],
                             mb.at[pl.ds(s * rptm, rptm)], ms)

        @pl.when(s == _NS - 1)
        def _():
            pltpu.async_copy(z_hbm.at[pl.ds(0, last)],
                             acc.at[pl.ds(s * rpt, last)], zs)
            pltpu.async_copy(m_hbm.at[pl.ds(s * rptm, lastm)],
                             mb.at[pl.ds(s * rptm, lastm)], ms)

        pltpu.sync_copy(src_hbm.at[pl.ds(w * jpt, jb)], si0)
        pltpu.sync_copy(dst_hbm.at[pl.ds(w * jpt, jb)], di0)

        @pl.when(s < _NS - 1)
        def _():
            pltpu.make_async_copy(z_hbm, acc.at[pl.ds(s * rpt, rpt)],
                                  zs).wait()
            pltpu.make_async_copy(m_hbm.at[pl.ds(s * rptm, rptm)],
                                  mb.at[pl.ds(s * rptm, rptm)], ms).wait()

        @pl.when(s == _NS - 1)
        def _():
            pltpu.make_async_copy(z_hbm.at[pl.ds(0, last)],
                                  acc.at[pl.ds(s * rpt, last)], zs).wait()
            pltpu.make_async_copy(m_hbm.at[pl.ds(s * rptm, lastm)],
                                  mb.at[pl.ds(s * rptm, lastm)], ms).wait()

        plsc.subcore_barrier()

        for bi in range(nb):
            sidx, didx = (si0, di0) if bi % 2 == 0 else (si1, di1)
            nsi, ndi = (si1, di1) if bi % 2 == 0 else (si0, di0)
            if bi + 1 < nb:
                row1 = w * jpt + (bi + 1) * jb
                pltpu.async_copy(src_hbm.at[pl.ds(row1, jb)], nsi, isem)
                pltpu.async_copy(dst_hbm.at[pl.ds(row1, jb)], ndi, isem)
            for l in range(3):
                pltpu.async_copy(mb.at[sidx.at[l]], bufs[l], gsems[l])

            def quad(t, carry, sidx=sidx, didx=didx):
                for l in range(4):
                    j = 4 * t + l
                    pltpu.make_async_copy(mb.at[sidx.at[j]],
                                          bufs[l], gsems[l]).wait()
                    pltpu.async_copy(bufs[l], acc.at[didx.at[j]],
                                     ssems[l], add=True)

                    @pl.when(j + 3 < jb)
                    def _():
                        ln = (l + 3) % 4

                        @pl.when(j > 0)
                        def _():
                            pltpu.make_async_copy(
                                bufs[ln], acc.at[didx.at[j]],
                                ssems[ln]).wait()

                        pltpu.async_copy(mb.at[sidx.at[j + 3]],
                                         bufs[ln], gsems[ln])
                return carry

            lax.fori_loop(0, jb // 4, quad, 0)
            for l in range(4):
                pltpu.make_async_copy(bufs[l], acc.at[didx.at[l]],
                                      ssems[l]).wait()
            if bi + 1 < nb:
                row1 = w * jpt + (bi + 1) * jb
                pltpu.make_async_copy(src_hbm.at[pl.ds(row1, jb)],
                                      nsi, isem).wait()
                pltpu.make_async_copy(dst_hbm.at[pl.ds(row1, jb)],
                                      ndi, isem).wait()

        plsc.subcore_barrier()

        @pl.when(s < _NS - 1)
        def _():
            pltpu.sync_copy(acc.at[pl.ds(s * rpt, rpt)],
                            out_hbm.at[c, pl.ds(s * rpt, rpt)])

        @pl.when(s == _NS - 1)
        def _():
            pltpu.sync_copy(acc.at[pl.ds(s * rpt, last)],
                            out_hbm.at[c, pl.ds(s * rpt, last)])

    return kern(m, src2, dst2, zeros32)


def _sc_gather(xb, idx, b, width):
    bpw = b // _NW
    mesh = plsc.VectorSubcoreMesh(core_axis_name="c", subcore_axis_name="s")

    @functools.partial(
        pl.kernel,
        out_type=jax.ShapeDtypeStruct((b, width), jnp.float32),
        mesh=mesh,
        compiler_params=pltpu.CompilerParams(use_tc_tiling_on_sc=False),
        scratch_types=[
            pltpu.VMEM((bpw,), jnp.int32),
            pltpu.VMEM((bpw, width), jnp.float32),
            pltpu.SemaphoreType.DMA,
        ],
    )
    def kern(x_hbm, idx_hbm, out_hbm, idxv, rows, sem):
        c = lax.axis_index("c")
        s = lax.axis_index("s")
        base = (s * _NC + c) * bpw
        pltpu.sync_copy(idx_hbm.at[pl.ds(base, bpw)], idxv)
        pltpu.async_copy(x_hbm.at[idxv], rows, sem).wait()
        pltpu.sync_copy(rows, out_hbm.at[pl.ds(base, bpw)])

    return kern(xb, idx)


def _row32(v):
    u = v % _BR
    return _BR * (v // _BR) + 4 * (u % _P) + u // _P


def _row16(v):
    u = v % _BR
    return 8 * (256 * (v // _BR) + u % 256) + u // 256


def kernel(node_features, params, edges, input_node_indices):
    n, d = node_features.shape
    e = edges.shape[1]
    inv_e = 1.0 / float(e)
    g = -(-n // _BR)
    n2 = g * _BR
    n_acc = n2 + 64

    ffns = {name: [_bn_params(l) for l in params[name]]
            for name in ("pre", "conv1_prep", "conv1_upd",
                         "conv2_prep", "conv2_upd", "post")}

    jpt = -(-e // (_NW * 128))
    jpt += jpt % 2
    e_tot = _NW * 128 * jpt
    dstp = jnp.concatenate(
        [edges[0], jnp.full((e_tot - e,), n2, jnp.int32)])
    srcp = jnp.concatenate(
        [edges[1], jnp.zeros((e_tot - e,), jnp.int32)])
    src2 = _row32(srcp).reshape(-1, 128)
    dst2 = _row32(dstp).reshape(-1, 128)
    rpt, _ = _row_partition(n_acc)
    zeros32b = jnp.zeros((rpt, 32), jnp.bfloat16)
    zeros32f = jnp.zeros((rpt, 32), jnp.float32)
    ones32 = jnp.ones((128, 32), jnp.float32)

    cnts = _sc_counts(dst2, zeros32f, ones32, n_acc, jpt)
    cnts_p = cnts.reshape(_NC, n_acc // 4, 128)

    x_p, m1_p = _tc1(node_features, ffns["pre"], ffns["conv1_prep"],
                     inv_e, n, g)
    sums1 = _sc_scatter(m1_p.reshape(n2, 32), src2, dst2, zeros32b,
                        n_acc, jpt)
    xa_p, m2_p, ccm = _tc_combine(x_p, sums1.reshape(_NC, n_acc // 4, 128),
                                  cnts_p, ffns["conv1_upd"],
                                  ffns["conv2_prep"], None, inv_e, g)
    sums2 = _sc_scatter(m2_p.reshape(n2, 32), src2, dst2, zeros32b,
                        n_acc, jpt)
    ncls = params["Wl"].shape[1]
    y_p = _tc_combine(xa_p, sums2.reshape(_NC, n_acc // 4, 128),
                      ccm, ffns["conv2_upd"], None,
                      (ffns["post"], params["Wl"],
                       params["bl"].reshape(1, -1)),
                      inv_e, g)[0]
    out = _sc_gather(y_p.reshape(n2, 32), _row32(input_node_indices),
                     input_node_indices.shape[0], 32)
    return out[:, :ncls]
